# Initial kernel scaffold; baseline (speedup 1.0000x reference)
import jax
import jax.numpy as jnp
from jax.experimental import pallas as pl


def kernel(edge_index, rels, scores, edge_conf_mask, edge_mask, mask, r_query_embed, conf_embeds, rel_embed_table, beta_w, beta_b, msg_w, msg_b, upd_w, upd_b, ln_g, ln_b):
    raise NotImplementedError("write your pallas kernel here")


# speedup vs baseline: 3.6472x; 3.6472x over previous
import functools

import jax
import jax.numpy as jnp
from jax import lax
from jax.experimental import pallas as pl
from jax.experimental.pallas import tpu as pltpu
from jax.experimental.pallas import tpu_sc as plsc

D = 128
TAU = 0.1
N_LAYERS = 3
E_PAD = 20480
N_PAD = 10240
NC = 2
NS = 16
CHUNK = 128
R = 512


def _sc_gather_rows(table, idx):
    M = idx.shape[0]
    per_tile = M // (NC * NS)
    n_chunks = per_tile // CHUNK
    mesh = plsc.VectorSubcoreMesh(core_axis_name="c", subcore_axis_name="s")

    @functools.partial(
        pl.kernel,
        out_type=jax.ShapeDtypeStruct((M, D), jnp.float32),
        mesh=mesh,
        scratch_types=[
            pltpu.VMEM((CHUNK,), jnp.int32),
            pltpu.VMEM((CHUNK, D), jnp.float32),
            pltpu.SemaphoreType.DMA,
        ],
    )
    def k(table_hbm, idx_hbm, out_hbm, idx_v, rows_v, sem):
        wid = lax.axis_index("s") * NC + lax.axis_index("c")
        base = wid * per_tile

        def body(j, _):
            off = base + j * CHUNK
            pltpu.sync_copy(idx_hbm.at[pl.ds(off, CHUNK)], idx_v)
            pltpu.async_copy(table_hbm.at[idx_v], rows_v, sem).wait()
            pltpu.sync_copy(rows_v, out_hbm.at[pl.ds(off, CHUNK)])
            return ()

        lax.fori_loop(0, n_chunks, body, ())

    return k(table, idx)


def _sc_scatter_add(msg, tgt, n_batch):
    per_tile = E_PAD // NS
    n_chunks = per_tile // CHUNK
    out_rows = N_PAD // NS
    n_out = out_rows // CHUNK
    mesh = plsc.VectorSubcoreMesh(core_axis_name="c", subcore_axis_name="s")

    @functools.partial(
        pl.kernel,
        out_type=jax.ShapeDtypeStruct((n_batch, N_PAD, D), jnp.float32),
        mesh=mesh,
        scratch_types=[
            pltpu.VMEM((CHUNK,), jnp.int32),
            pltpu.VMEM((CHUNK, D), jnp.float32),
            pltpu.VMEM((CHUNK, D), jnp.float32),
            pltpu.VMEM_SHARED((N_PAD, D), jnp.float32),
            pltpu.SemaphoreType.DMA,
        ],
    )
    def k(msg_hbm, tgt_hbm, out_hbm, idx_v, zero_v, rows_v, aggr_sp, sem):
        c = lax.axis_index("c")
        s = lax.axis_index("s")

        def zrow(i, _):
            for t in range(D // 16):
                zero_v[i, pl.ds(t * 16, 16)] = jnp.zeros((16,), jnp.float32)
            return ()

        lax.fori_loop(0, CHUNK, zrow, ())

        for bi in range(n_batch // NC):
            b = bi * NC + c

            def zbody(j, _):
                pltpu.sync_copy(
                    zero_v, aggr_sp.at[pl.ds(s * out_rows + j * CHUNK, CHUNK)])
                return ()

            lax.fori_loop(0, n_out, zbody, ())
            plsc.subcore_barrier()

            def sbody(j, _):
                off = b * E_PAD + s * per_tile + j * CHUNK
                pltpu.sync_copy(tgt_hbm.at[pl.ds(off, CHUNK)], idx_v)
                pltpu.sync_copy(msg_hbm.at[pl.ds(off, CHUNK)], rows_v)
                pltpu.sync_copy(rows_v, aggr_sp.at[idx_v], add=True)
                return ()

            lax.fori_loop(0, n_chunks, sbody, ())
            plsc.subcore_barrier()

            def obody(j, _):
                r0 = s * out_rows + j * CHUNK
                pltpu.sync_copy(aggr_sp.at[pl.ds(r0, CHUNK)], rows_v)
                pltpu.sync_copy(rows_v, out_hbm.at[b, pl.ds(r0, CHUNK)])
                return ()

            lax.fori_loop(0, n_out, obody, ())
            plsc.subcore_barrier()

    return k(msg, tgt)


def _edge_specs(n_batch):
    feat = pl.BlockSpec((1, R, D), lambda b, g: (b, g, 0))
    col = pl.BlockSpec((1, R, 1), lambda b, g: (b, g, 0))
    full2 = lambda shape: pl.BlockSpec(shape, lambda b, g: (0, 0))
    grid = (n_batch, E_PAD // R)
    return feat, col, full2, grid


def _msg0_body(hr_ref, cf_ref, sc_ref, cm_ref, em_ref, src_ref, rq_ref,
               bw_ref, bb_ref, w1_ref, wc_ref, v0_ref, b0_ref,
               msg_ref, gate_ref):
    hr = hr_ref[0]
    cf = cf_ref[0]
    bwr = bw_ref[...]
    logit = (jnp.sum(hr * bwr, axis=1, keepdims=True)
             + jnp.sum(rq_ref[0] * bwr, axis=1, keepdims=True) + bb_ref[0, 0])
    beta = jax.nn.sigmoid(logit)
    gk = jax.nn.sigmoid((sc_ref[0] - beta) / TAU)
    gate = jnp.where(cm_ref[0] > 0, gk, 0.5) * em_ref[0]
    gate_ref[0] = gate
    is0 = (src_ref[0] == 0).astype(jnp.float32)
    t1 = jnp.dot(hr, w1_ref[...], preferred_element_type=jnp.float32)
    a2 = jnp.dot(jnp.concatenate([hr, cf], axis=1), wc_ref[...],
                 preferred_element_type=jnp.float32)
    z = is0 * (t1 + v0_ref[...]) + a2 + b0_ref[...]
    msg_ref[0] = gate * jnp.maximum(z, 0.0)


def _msgk_body(hs_ref, hr_ref, cf_ref, gate_ref, src_ref,
               wc_ref, w3s_ref, bk_ref, msg_ref):
    hs = hs_ref[0]
    hr = hr_ref[0]
    a = jnp.concatenate([hs * hr, hs, hr, cf_ref[0]], axis=1)
    z = jnp.dot(a, wc_ref[...], preferred_element_type=jnp.float32)
    is0 = (src_ref[0] == 0).astype(jnp.float32)
    z = z + is0 * w3s_ref[...] + bk_ref[...]
    msg_ref[0] = gate_ref[0] * jnp.maximum(z, 0.0)


def _msg2_body(hs_ref, hr_ref, cf_ref, gate_ref, src_ref, tgt_ref,
               wc_ref, w3s_ref, bk_ref, out_ref):
    g = pl.program_id(1)
    hs = hs_ref[0]
    hr = hr_ref[0]
    a = jnp.concatenate([hs * hr, hs, hr, cf_ref[0]], axis=1)
    z = jnp.dot(a, wc_ref[...], preferred_element_type=jnp.float32)
    is0 = (src_ref[0] == 0).astype(jnp.float32)
    z = z + is0 * w3s_ref[...] + bk_ref[...]
    wmsg = gate_ref[0] * jnp.maximum(z, 0.0)
    t0 = (tgt_ref[0] == 0).astype(jnp.float32)
    part = jnp.sum(wmsg * t0, axis=0, keepdims=True)

    @pl.when(g == 0)
    def _():
        out_ref[0] = part

    @pl.when(g > 0)
    def _():
        out_ref[0] = out_ref[0] + part


def _upd_body(ag_ref, h_ref, w_ref, ub_ref, lg_ref, lb_ref, out_ref):
    u = jnp.dot(ag_ref[0], w_ref[...],
                preferred_element_type=jnp.float32) + ub_ref[...]
    x = h_ref[0] + u
    m = jnp.mean(x, axis=1, keepdims=True)
    xc = x - m
    v = jnp.mean(xc * xc, axis=1, keepdims=True)
    out_ref[0] = xc * lax.rsqrt(v + 1e-5) * lg_ref[...] + lb_ref[...]


def _final_body(a0_ref, h0_ref, w_ref, ub_ref, lg_ref, lb_ref, out_ref):
    u = jnp.dot(a0_ref[:, 0, :], w_ref[...],
                preferred_element_type=jnp.float32) + ub_ref[...]
    x = h0_ref[:, 0, :] + u
    m = jnp.mean(x, axis=1, keepdims=True)
    xc = x - m
    v = jnp.mean(xc * xc, axis=1, keepdims=True)
    out_ref[:, 0, :] = xc * lax.rsqrt(v + 1e-5) * lg_ref[...] + lb_ref[...]


def kernel(edge_index, rels, scores, edge_conf_mask, edge_mask, mask,
           r_query_embed, conf_embeds, rel_embed_table, beta_w, beta_b,
           msg_w, msg_b, upd_w, upd_b, ln_g, ln_b):
    Bn, MaxN = mask.shape
    E = rels.shape[1]
    pe = E_PAD - E

    src = jnp.pad(edge_index[:, 0, :], ((0, 0), (0, pe)))
    tgt = jnp.pad(edge_index[:, 1, :], ((0, 0), (0, pe)))
    rels_f = jnp.pad(rels, ((0, 0), (0, pe))).reshape(-1)
    sc_c = jnp.pad(scores, ((0, 0), (0, pe)))[..., None]
    cm_c = jnp.pad(edge_conf_mask.astype(jnp.int32), ((0, 0), (0, pe)))[..., None]
    em_c = jnp.pad(edge_mask.astype(jnp.float32), ((0, 0), (0, pe)))[..., None]
    src_c = src[..., None]
    tgt_c = tgt[..., None]
    cf = jnp.pad(conf_embeds, ((0, 0), (0, pe), (0, 0)))
    src_g = (src + jnp.arange(Bn, dtype=jnp.int32)[:, None] * N_PAD).reshape(-1)
    tgt_f = tgt.reshape(-1)
    rq = r_query_embed[:, None, :]
    bw_row = beta_w.reshape(1, D)
    bb = beta_b.reshape(1, 1)

    w1 = msg_w[:, 0:D, :]
    w2s = jnp.sum(msg_w[:, D:2 * D, :], axis=1, keepdims=True)
    w3s = jnp.sum(msg_w[:, 2 * D:3 * D, :], axis=1, keepdims=True)
    w45 = msg_w[:, 3 * D:5 * D, :]
    wcat = jnp.concatenate([w1, msg_w[:, D:2 * D, :], w45], axis=1)
    mb = msg_b[:, None, :]
    ub = upd_b[:, None, :]
    lg = ln_g.reshape(1, D)
    lb = ln_b.reshape(1, D)

    h0 = jnp.zeros((Bn, N_PAD, D), jnp.float32).at[:, 0, :].set(1.0)

    feat, col, full2, grid = _edge_specs(Bn)
    node = pl.BlockSpec((1, 1024, D), lambda b, g: (b, g, 0))
    node_grid = (Bn, N_PAD // 1024)
    rq_spec = pl.BlockSpec((1, 1, D), lambda b, g: (b, 0, 0))

    hr = _sc_gather_rows(rel_embed_table, rels_f).reshape(Bn, E_PAD, D)

    msg0, gate = pl.pallas_call(
        _msg0_body,
        grid=grid,
        in_specs=[feat, feat, col, col, col, col, rq_spec,
                  full2((1, D)), full2((1, 1)), full2((D, D)),
                  full2((2 * D, D)), full2((1, D)), full2((1, D))],
        out_specs=[feat, col],
        out_shape=[jax.ShapeDtypeStruct((Bn, E_PAD, D), jnp.float32),
                   jax.ShapeDtypeStruct((Bn, E_PAD, 1), jnp.float32)],
    )(hr, cf, sc_c, cm_c, em_c, src_c, rq, bw_row, bb,
      w1[0], w45[0], w2s[0] + w3s[0], mb[0])

    def run_update(aggr, h_prev, k):
        return pl.pallas_call(
            _upd_body,
            grid=node_grid,
            in_specs=[node, node, full2((D, D)), full2((1, D)),
                      full2((1, D)), full2((1, D))],
            out_specs=node,
            out_shape=jax.ShapeDtypeStruct((Bn, N_PAD, D), jnp.float32),
        )(aggr, h_prev, upd_w[k], ub[k], lg, lb)

    aggr0 = _sc_scatter_add(msg0.reshape(-1, D), tgt_f, Bn)
    h1 = run_update(aggr0, h0, 0)

    hs1 = _sc_gather_rows(h1.reshape(-1, D), src_g).reshape(Bn, E_PAD, D)
    msg1 = pl.pallas_call(
        _msgk_body,
        grid=grid,
        in_specs=[feat, feat, feat, col, col,
                  full2((4 * D, D)), full2((1, D)), full2((1, D))],
        out_specs=feat,
        out_shape=jax.ShapeDtypeStruct((Bn, E_PAD, D), jnp.float32),
    )(hs1, hr, cf, gate, src_c, wcat[1], w3s[1], mb[1])
    aggr1 = _sc_scatter_add(msg1.reshape(-1, D), tgt_f, Bn)
    h2 = run_update(aggr1, h1, 1)

    hs2 = _sc_gather_rows(h2.reshape(-1, D), src_g).reshape(Bn, E_PAD, D)
    a0 = pl.pallas_call(
        _msg2_body,
        grid=grid,
        in_specs=[feat, feat, feat, col, col, col,
                  full2((4 * D, D)), full2((1, D)), full2((1, D))],
        out_specs=pl.BlockSpec((1, 1, D), lambda b, g: (b, 0, 0)),
        out_shape=jax.ShapeDtypeStruct((Bn, 1, D), jnp.float32),
    )(hs2, hr, cf, gate, src_c, tgt_c, wcat[2], w3s[2], mb[2])

    full3 = lambda shape: pl.BlockSpec(shape, lambda: (0,) * len(shape))
    ctx2 = pl.pallas_call(
        _final_body,
        in_specs=[full3((Bn, 1, D)), full3((Bn, 1, D)), full3((D, D)),
                  full3((1, D)), full3((1, D)), full3((1, D))],
        out_specs=full3((Bn, 1, D)),
        out_shape=jax.ShapeDtypeStruct((Bn, 1, D), jnp.float32),
    )(a0, h2[:, :1, :], upd_w[2], ub[2], lg, lb)

    return jnp.stack([h1[:, 0, :], h2[:, 0, :], ctx2[:, 0, :]], axis=1)


# speedup vs baseline: 3.8953x; 1.0680x over previous
import functools

import jax
import jax.numpy as jnp
from jax import lax
from jax.experimental import pallas as pl
from jax.experimental.pallas import tpu as pltpu
from jax.experimental.pallas import tpu_sc as plsc

D = 128
TAU = 0.1
N_LAYERS = 3
E_PAD = 20480
N_PAD = 10240
NC = 2
NS = 16
CHUNK = 128
R = 512


def _sc_gather_rows(table, idx, ncol=D):
    M = idx.shape[0]
    dtype = table.dtype
    per_tile = M // (NC * NS)
    n_chunks = per_tile // CHUNK
    mesh = plsc.VectorSubcoreMesh(core_axis_name="c", subcore_axis_name="s")

    @functools.partial(
        pl.kernel,
        out_type=jax.ShapeDtypeStruct((M, ncol), dtype),
        mesh=mesh,
        scratch_types=[
            pltpu.VMEM((per_tile,), jnp.int32),
            pltpu.VMEM((CHUNK, ncol), dtype),
            pltpu.VMEM((CHUNK, ncol), dtype),
            pltpu.SemaphoreType.DMA,
            pltpu.SemaphoreType.DMA,
        ],
    )
    def k(table_hbm, idx_hbm, out_hbm, idx_all, rows0, rows1, gsem, wsem):
        wid = lax.axis_index("s") * NC + lax.axis_index("c")
        base = wid * per_tile
        pltpu.sync_copy(idx_hbm.at[pl.ds(base, per_tile)], idx_all)
        rows = (rows0, rows1)
        g = [None] * n_chunks
        w = [None] * n_chunks
        g[0] = pltpu.async_copy(
            table_hbm.at[idx_all.at[pl.ds(0, CHUNK)]], rows0, gsem)
        for j in range(n_chunks):
            g[j].wait()
            if j + 1 < n_chunks:
                if j >= 1:
                    w[j - 1].wait()
                g[j + 1] = pltpu.async_copy(
                    table_hbm.at[idx_all.at[pl.ds((j + 1) * CHUNK, CHUNK)]],
                    rows[(j + 1) % 2], gsem)
            w[j] = pltpu.async_copy(
                rows[j % 2], out_hbm.at[pl.ds(base + j * CHUNK, CHUNK)], wsem)
        if n_chunks >= 2:
            w[n_chunks - 2].wait()
        w[n_chunks - 1].wait()

    return k(table, idx)


def _sc_scatter_add(msg, tgt, n_batch):
    per_tile = E_PAD // NS
    n_chunks = per_tile // CHUNK
    out_rows = N_PAD // NS
    n_out = out_rows // CHUNK
    mesh = plsc.VectorSubcoreMesh(core_axis_name="c", subcore_axis_name="s")

    @functools.partial(
        pl.kernel,
        out_type=jax.ShapeDtypeStruct((n_batch, N_PAD, D), jnp.float32),
        mesh=mesh,
        scratch_types=[
            pltpu.VMEM((CHUNK,), jnp.int32),
            pltpu.VMEM((CHUNK,), jnp.int32),
            pltpu.VMEM((CHUNK, D), jnp.float32),
            pltpu.VMEM((CHUNK, D), jnp.float32),
            pltpu.VMEM((CHUNK // 2, D), jnp.float32),
            pltpu.VMEM_SHARED((N_PAD, D), jnp.float32),
            pltpu.SemaphoreType.DMA,
            pltpu.SemaphoreType.DMA,
            pltpu.SemaphoreType.DMA,
        ],
    )
    def k(msg_hbm, tgt_hbm, out_hbm, ib0, ib1, mb0, mb1, zero_v, aggr_sp,
          lsem, zsem, wsem):
        c = lax.axis_index("c")
        s = lax.axis_index("s")
        ib = (ib0, ib1)
        mb = (mb0, mb1)

        def zrow(i, _):
            for t in range(D // 16):
                zero_v[i, pl.ds(t * 16, 16)] = jnp.zeros((16,), jnp.float32)
            return ()

        lax.fori_loop(0, CHUNK // 2, zrow, ())

        for bi in range(n_batch // NC):
            b = bi * NC + c
            ebase = b * E_PAD + s * per_tile
            nbase = s * out_rows

            zs = [pltpu.async_copy(
                zero_v,
                aggr_sp.at[pl.ds(nbase + j * (CHUNK // 2), CHUNK // 2)], zsem)
                for j in range(2 * n_out)]
            for d in zs:
                d.wait()
            plsc.subcore_barrier()

            il = [None] * n_chunks
            ml = [None] * n_chunks
            il[0] = pltpu.async_copy(
                tgt_hbm.at[pl.ds(ebase, CHUNK)], ib0, lsem)
            ml[0] = pltpu.async_copy(
                msg_hbm.at[pl.ds(ebase, CHUNK)], mb0, lsem)
            for j in range(n_chunks):
                il[j].wait()
                ml[j].wait()
                if j + 1 < n_chunks:
                    off = ebase + (j + 1) * CHUNK
                    il[j + 1] = pltpu.async_copy(
                        tgt_hbm.at[pl.ds(off, CHUNK)], ib[(j + 1) % 2], lsem)
                    ml[j + 1] = pltpu.async_copy(
                        msg_hbm.at[pl.ds(off, CHUNK)], mb[(j + 1) % 2], lsem)
                pltpu.sync_copy(mb[j % 2], aggr_sp.at[ib[j % 2]], add=True)
            plsc.subcore_barrier()

            wb = [None] * n_out
            for j in range(n_out):
                if j >= 2:
                    wb[j - 2].wait()
                r0 = nbase + j * CHUNK
                pltpu.sync_copy(aggr_sp.at[pl.ds(r0, CHUNK)], mb[j % 2])
                wb[j] = pltpu.async_copy(
                    mb[j % 2], out_hbm.at[b, pl.ds(r0, CHUNK)], wsem)
            wb[n_out - 2].wait()
            wb[n_out - 1].wait()
            plsc.subcore_barrier()

    return k(msg, tgt)


def _edge_specs(n_batch):
    feat = pl.BlockSpec((1, R, D), lambda b, g: (b, g, 0))
    col = pl.BlockSpec((1, R, 1), lambda b, g: (b, g, 0))
    full2 = lambda shape: pl.BlockSpec(shape, lambda b, g: (0, 0))
    grid = (n_batch, E_PAD // R)
    return feat, col, full2, grid


def _msg0_body(hr_ref, cf_ref, sc_ref, cm_ref, em_ref, src_ref, rq_ref,
               bw_ref, bb_ref, w14_ref, w5_ref, v0_ref, b0_ref,
               msg_ref, gate_ref, hrbf_ref):
    hr = hr_ref[0]
    cf = cf_ref[0]
    hrbf_ref[0] = hr.astype(jnp.bfloat16)
    bwr = bw_ref[...]
    logit = (jnp.sum(hr * bwr, axis=1, keepdims=True)
             + jnp.sum(rq_ref[0] * bwr, axis=1, keepdims=True) + bb_ref[0, 0])
    beta = jax.nn.sigmoid(logit)
    gk = jax.nn.sigmoid((sc_ref[0] - beta) / TAU)
    gate = jnp.where(cm_ref[0] > 0, gk, 0.5) * em_ref[0]
    gate_ref[0] = gate
    is0 = (src_ref[0] == 0).astype(jnp.float32)
    t14 = jnp.dot(hr, w14_ref[...], preferred_element_type=jnp.float32)
    t5 = jnp.dot(cf, w5_ref[...], preferred_element_type=jnp.float32)
    z = is0 * (t14[:, :D] + v0_ref[...]) + t14[:, D:] + t5 + b0_ref[...]
    msg_ref[0] = gate * jnp.maximum(z, 0.0)


def _msgk_body(hs_ref, hr_ref, cf_ref, gate_ref, src_ref,
               w12_ref, w45_ref, w3s_ref, bk_ref, msg_ref):
    hs = hs_ref[0]
    hr = hr_ref[0]
    a = jnp.concatenate([hs * hr.astype(jnp.float32), hs], axis=1)
    b = jnp.concatenate([hr, cf_ref[0]], axis=1)
    z = (jnp.dot(a, w12_ref[...], preferred_element_type=jnp.float32)
         + jnp.dot(b, w45_ref[...], preferred_element_type=jnp.float32))
    is0 = (src_ref[0] == 0).astype(jnp.float32)
    z = z + is0 * w3s_ref[...] + bk_ref[...]
    msg_ref[0] = gate_ref[0] * jnp.maximum(z, 0.0)


def _msg2_body(hs_ref, hr_ref, cf_ref, gate_ref, src_ref, tgt_ref,
               w12_ref, w45_ref, w3s_ref, bk_ref, out_ref):
    g = pl.program_id(1)
    hs = hs_ref[0]
    hr = hr_ref[0]
    a = jnp.concatenate([hs * hr.astype(jnp.float32), hs], axis=1)
    b = jnp.concatenate([hr, cf_ref[0]], axis=1)
    z = (jnp.dot(a, w12_ref[...], preferred_element_type=jnp.float32)
         + jnp.dot(b, w45_ref[...], preferred_element_type=jnp.float32))
    is0 = (src_ref[0] == 0).astype(jnp.float32)
    z = z + is0 * w3s_ref[...] + bk_ref[...]
    wmsg = gate_ref[0] * jnp.maximum(z, 0.0)
    t0 = (tgt_ref[0] == 0).astype(jnp.float32)
    part = jnp.sum(wmsg * t0, axis=0, keepdims=True)

    @pl.when(g == 0)
    def _():
        out_ref[0] = part

    @pl.when(g > 0)
    def _():
        out_ref[0] = out_ref[0] + part


def _upd_body(ag_ref, h_ref, w_ref, ub_ref, lg_ref, lb_ref, out_ref):
    u = jnp.dot(ag_ref[0], w_ref[...],
                preferred_element_type=jnp.float32) + ub_ref[...]
    x = h_ref[0] + u
    m = jnp.mean(x, axis=1, keepdims=True)
    xc = x - m
    v = jnp.mean(xc * xc, axis=1, keepdims=True)
    out_ref[0] = xc * lax.rsqrt(v + 1e-5) * lg_ref[...] + lb_ref[...]


def _final_body(a0_ref, h0_ref, w_ref, ub_ref, lg_ref, lb_ref, out_ref):
    u = jnp.dot(a0_ref[:, 0, :], w_ref[...],
                preferred_element_type=jnp.float32) + ub_ref[...]
    x = h0_ref[:, 0, :] + u
    m = jnp.mean(x, axis=1, keepdims=True)
    xc = x - m
    v = jnp.mean(xc * xc, axis=1, keepdims=True)
    out_ref[:, 0, :] = xc * lax.rsqrt(v + 1e-5) * lg_ref[...] + lb_ref[...]


def kernel(edge_index, rels, scores, edge_conf_mask, edge_mask, mask,
           r_query_embed, conf_embeds, rel_embed_table, beta_w, beta_b,
           msg_w, msg_b, upd_w, upd_b, ln_g, ln_b):
    Bn, MaxN = mask.shape
    E = rels.shape[1]
    pe = E_PAD - E

    src = jnp.pad(edge_index[:, 0, :], ((0, 0), (0, pe)))
    tgt = jnp.pad(edge_index[:, 1, :], ((0, 0), (0, pe)))
    rels_f = jnp.pad(rels, ((0, 0), (0, pe))).reshape(-1)
    sc_c = jnp.pad(scores, ((0, 0), (0, pe)))[..., None]
    cm_c = jnp.pad(edge_conf_mask.astype(jnp.int32), ((0, 0), (0, pe)))[..., None]
    em_c = jnp.pad(edge_mask.astype(jnp.float32), ((0, 0), (0, pe)))[..., None]
    src_c = src[..., None]
    tgt_c = tgt[..., None]
    cf = jnp.pad(conf_embeds, ((0, 0), (0, pe), (0, 0))).astype(jnp.bfloat16)
    src_g = (src + jnp.arange(Bn, dtype=jnp.int32)[:, None] * N_PAD).reshape(-1)
    tgt_f = tgt.reshape(-1)
    rq = r_query_embed[:, None, :]
    bw_row = beta_w.reshape(1, D)
    bb = beta_b.reshape(1, 1)

    w1 = msg_w[:, 0:D, :]
    w2s = jnp.sum(msg_w[:, D:2 * D, :], axis=1, keepdims=True)
    w3s = jnp.sum(msg_w[:, 2 * D:3 * D, :], axis=1, keepdims=True)
    w14_0 = jnp.concatenate([w1[0], msg_w[0, 3 * D:4 * D, :]], axis=1)
    w5_0 = msg_w[0, 4 * D:5 * D, :].astype(jnp.bfloat16)
    w12 = jnp.concatenate([w1, msg_w[:, D:2 * D, :]], axis=1)
    w45bf = msg_w[:, 3 * D:5 * D, :].astype(jnp.bfloat16)
    mb = msg_b[:, None, :]
    ub = upd_b[:, None, :]
    lg = ln_g.reshape(1, D)
    lb = ln_b.reshape(1, D)

    h0 = jnp.zeros((Bn, N_PAD, D), jnp.float32).at[:, 0, :].set(1.0)

    feat, col, full2, grid = _edge_specs(Bn)
    node = pl.BlockSpec((1, 1024, D), lambda b, g: (b, g, 0))
    node_grid = (Bn, N_PAD // 1024)
    rq_spec = pl.BlockSpec((1, 1, D), lambda b, g: (b, 0, 0))

    hr = _sc_gather_rows(rel_embed_table, rels_f).reshape(Bn, E_PAD, D)

    msg0, gate, hr_bf = pl.pallas_call(
        _msg0_body,
        grid=grid,
        in_specs=[feat, feat, col, col, col, col, rq_spec,
                  full2((1, D)), full2((1, 1)), full2((D, 2 * D)),
                  full2((D, D)), full2((1, D)), full2((1, D))],
        out_specs=[feat, col, feat],
        out_shape=[jax.ShapeDtypeStruct((Bn, E_PAD, D), jnp.float32),
                   jax.ShapeDtypeStruct((Bn, E_PAD, 1), jnp.float32),
                   jax.ShapeDtypeStruct((Bn, E_PAD, D), jnp.bfloat16)],
    )(hr, cf, sc_c, cm_c, em_c, src_c, rq, bw_row, bb,
      w14_0, w5_0, w2s[0] + w3s[0], mb[0])

    def run_update(aggr, h_prev, k):
        return pl.pallas_call(
            _upd_body,
            grid=node_grid,
            in_specs=[node, node, full2((D, D)), full2((1, D)),
                      full2((1, D)), full2((1, D))],
            out_specs=node,
            out_shape=jax.ShapeDtypeStruct((Bn, N_PAD, D), jnp.float32),
        )(aggr, h_prev, upd_w[k], ub[k], lg, lb)

    aggr0 = _sc_scatter_add(msg0.reshape(-1, D), tgt_f, Bn)
    h1 = run_update(aggr0, h0, 0)

    hs1 = _sc_gather_rows(h1.reshape(-1, D), src_g).reshape(Bn, E_PAD, D)
    msg1 = pl.pallas_call(
        _msgk_body,
        grid=grid,
        in_specs=[feat, feat, feat, col, col,
                  full2((2 * D, D)), full2((2 * D, D)), full2((1, D)),
                  full2((1, D))],
        out_specs=feat,
        out_shape=jax.ShapeDtypeStruct((Bn, E_PAD, D), jnp.float32),
    )(hs1, hr_bf, cf, gate, src_c, w12[1], w45bf[1], w3s[1], mb[1])
    aggr1 = _sc_scatter_add(msg1.reshape(-1, D), tgt_f, Bn)
    h2 = run_update(aggr1, h1, 1)

    hs2 = _sc_gather_rows(h2.reshape(-1, D), src_g).reshape(Bn, E_PAD, D)
    a0 = pl.pallas_call(
        _msg2_body,
        grid=grid,
        in_specs=[feat, feat, feat, col, col, col,
                  full2((2 * D, D)), full2((2 * D, D)), full2((1, D)),
                  full2((1, D))],
        out_specs=pl.BlockSpec((1, 1, D), lambda b, g: (b, 0, 0)),
        out_shape=jax.ShapeDtypeStruct((Bn, 1, D), jnp.float32),
    )(hs2, hr_bf, cf, gate, src_c, tgt_c, w12[2], w45bf[2], w3s[2], mb[2])

    full3 = lambda shape: pl.BlockSpec(shape, lambda: (0,) * len(shape))
    ctx2 = pl.pallas_call(
        _final_body,
        in_specs=[full3((Bn, 1, D)), full3((Bn, 1, D)), full3((D, D)),
                  full3((1, D)), full3((1, D)), full3((1, D))],
        out_specs=full3((Bn, 1, D)),
        out_shape=jax.ShapeDtypeStruct((Bn, 1, D), jnp.float32),
    )(a0, h2[:, :1, :], upd_w[2], ub[2], lg, lb)

    return jnp.stack([h1[:, 0, :], h2[:, 0, :], ctx2[:, 0, :]], axis=1)


# speedup vs baseline: 4.1000x; 1.0525x over previous
import functools

import jax
import jax.numpy as jnp
from jax import lax
from jax.experimental import pallas as pl
from jax.experimental.pallas import tpu as pltpu
from jax.experimental.pallas import tpu_sc as plsc

D = 128
N_RELS = 512
TAU = 0.1
N_LAYERS = 3
E_PAD = 20480
N_PAD = 10240
NC = 2
NS = 16
CHUNK = 128
R = 512


def _sc_gather_rows(table, idx, ncol=D):
    M = idx.shape[0]
    dtype = table.dtype
    per_tile = M // (NC * NS)
    n_chunks = per_tile // CHUNK
    mesh = plsc.VectorSubcoreMesh(core_axis_name="c", subcore_axis_name="s")

    @functools.partial(
        pl.kernel,
        out_type=jax.ShapeDtypeStruct((M, ncol), dtype),
        mesh=mesh,
        scratch_types=[
            pltpu.VMEM((per_tile,), jnp.int32),
            pltpu.VMEM((CHUNK, ncol), dtype),
            pltpu.VMEM((CHUNK, ncol), dtype),
            pltpu.SemaphoreType.DMA,
            pltpu.SemaphoreType.DMA,
        ],
    )
    def k(table_hbm, idx_hbm, out_hbm, idx_all, rows0, rows1, gsem, wsem):
        wid = lax.axis_index("s") * NC + lax.axis_index("c")
        base = wid * per_tile
        pltpu.sync_copy(idx_hbm.at[pl.ds(base, per_tile)], idx_all)
        rows = (rows0, rows1)
        g = [None] * n_chunks
        w = [None] * n_chunks
        g[0] = pltpu.async_copy(
            table_hbm.at[idx_all.at[pl.ds(0, CHUNK)]], rows0, gsem)
        for j in range(n_chunks):
            g[j].wait()
            if j + 1 < n_chunks:
                if j >= 1:
                    w[j - 1].wait()
                g[j + 1] = pltpu.async_copy(
                    table_hbm.at[idx_all.at[pl.ds((j + 1) * CHUNK, CHUNK)]],
                    rows[(j + 1) % 2], gsem)
            w[j] = pltpu.async_copy(
                rows[j % 2], out_hbm.at[pl.ds(base + j * CHUNK, CHUNK)], wsem)
        if n_chunks >= 2:
            w[n_chunks - 2].wait()
        w[n_chunks - 1].wait()

    return k(table, idx)


def _sc_scatter_add(msg, tgt, n_batch):
    per_tile = E_PAD // NS
    n_chunks = per_tile // CHUNK
    out_rows = N_PAD // NS
    n_out = out_rows // CHUNK
    mesh = plsc.VectorSubcoreMesh(core_axis_name="c", subcore_axis_name="s")

    @functools.partial(
        pl.kernel,
        out_type=jax.ShapeDtypeStruct((n_batch, N_PAD, D), jnp.float32),
        mesh=mesh,
        scratch_types=[
            pltpu.VMEM((CHUNK,), jnp.int32),
            pltpu.VMEM((CHUNK,), jnp.int32),
            pltpu.VMEM((CHUNK, D), jnp.float32),
            pltpu.VMEM((CHUNK, D), jnp.float32),
            pltpu.VMEM((CHUNK // 2, D), jnp.float32),
            pltpu.VMEM_SHARED((N_PAD, D), jnp.float32),
            pltpu.SemaphoreType.DMA,
            pltpu.SemaphoreType.DMA,
            pltpu.SemaphoreType.DMA,
        ],
    )
    def k(msg_hbm, tgt_hbm, out_hbm, ib0, ib1, mb0, mb1, zero_v, aggr_sp,
          lsem, zsem, wsem):
        c = lax.axis_index("c")
        s = lax.axis_index("s")
        ib = (ib0, ib1)
        mb = (mb0, mb1)

        def zrow(i, _):
            for t in range(D // 16):
                zero_v[i, pl.ds(t * 16, 16)] = jnp.zeros((16,), jnp.float32)
            return ()

        lax.fori_loop(0, CHUNK // 2, zrow, ())

        for bi in range(n_batch // NC):
            b = bi * NC + c
            ebase = b * E_PAD + s * per_tile
            nbase = s * out_rows

            zs = [pltpu.async_copy(
                zero_v,
                aggr_sp.at[pl.ds(nbase + j * (CHUNK // 2), CHUNK // 2)], zsem)
                for j in range(2 * n_out)]
            for d in zs:
                d.wait()
            plsc.subcore_barrier()

            il = [None] * n_chunks
            ml = [None] * n_chunks
            il[0] = pltpu.async_copy(
                tgt_hbm.at[pl.ds(ebase, CHUNK)], ib0, lsem)
            ml[0] = pltpu.async_copy(
                msg_hbm.at[pl.ds(ebase, CHUNK)], mb0, lsem)
            for j in range(n_chunks):
                il[j].wait()
                ml[j].wait()
                if j + 1 < n_chunks:
                    off = ebase + (j + 1) * CHUNK
                    il[j + 1] = pltpu.async_copy(
                        tgt_hbm.at[pl.ds(off, CHUNK)], ib[(j + 1) % 2], lsem)
                    ml[j + 1] = pltpu.async_copy(
                        msg_hbm.at[pl.ds(off, CHUNK)], mb[(j + 1) % 2], lsem)
                pltpu.sync_copy(mb[j % 2], aggr_sp.at[ib[j % 2]], add=True)
            plsc.subcore_barrier()

            wb = [None] * n_out
            for j in range(n_out):
                if j >= 2:
                    wb[j - 2].wait()
                r0 = nbase + j * CHUNK
                pltpu.sync_copy(aggr_sp.at[pl.ds(r0, CHUNK)], mb[j % 2])
                wb[j] = pltpu.async_copy(
                    mb[j % 2], out_hbm.at[b, pl.ds(r0, CHUNK)], wsem)
            wb[n_out - 2].wait()
            wb[n_out - 1].wait()
            plsc.subcore_barrier()

    return k(msg, tgt)


def _edge_specs(n_batch):
    feat = pl.BlockSpec((1, R, D), lambda b, g: (b, g, 0))
    col = pl.BlockSpec((1, R, 1), lambda b, g: (b, g, 0))
    full2 = lambda shape: pl.BlockSpec(shape, lambda b, g: (0, 0))
    grid = (n_batch, E_PAD // R)
    return feat, col, full2, grid


def _msg0_body(hr_ref, cf_ref, sc_ref, cm_ref, em_ref, src_ref, rq_ref,
               bw_ref, bb_ref, w14_ref, w5_ref, v0_ref, b0_ref,
               msg_ref, gate_ref, hrbf_ref):
    hr = hr_ref[0]
    cf = cf_ref[0]
    hrbf_ref[0] = hr.astype(jnp.bfloat16)
    bwr = bw_ref[...]
    logit = (jnp.sum(hr * bwr, axis=1, keepdims=True)
             + jnp.sum(rq_ref[0] * bwr, axis=1, keepdims=True) + bb_ref[0, 0])
    beta = jax.nn.sigmoid(logit)
    gk = jax.nn.sigmoid((sc_ref[0] - beta) / TAU)
    gate = jnp.where(cm_ref[0] > 0, gk, 0.5) * em_ref[0]
    gate_ref[0] = gate
    is0 = (src_ref[0] == 0).astype(jnp.float32)
    t14 = jnp.dot(hr, w14_ref[...], preferred_element_type=jnp.float32)
    t5 = jnp.dot(cf, w5_ref[...], preferred_element_type=jnp.float32)
    z = is0 * (t14[:, :D] + v0_ref[...]) + t14[:, D:] + t5 + b0_ref[...]
    msg_ref[0] = gate * jnp.maximum(z, 0.0)


def _msgk_body(hs_ref, hr_ref, cf_ref, gate_ref, src_ref,
               w12_ref, w45_ref, w3s_ref, bk_ref, msg_ref):
    hs = hs_ref[0]
    hr = hr_ref[0]
    a = jnp.concatenate([hs * hr.astype(jnp.float32), hs], axis=1)
    b = jnp.concatenate([hr, cf_ref[0]], axis=1)
    z = (jnp.dot(a, w12_ref[...], preferred_element_type=jnp.float32)
         + jnp.dot(b, w45_ref[...], preferred_element_type=jnp.float32))
    is0 = (src_ref[0] == 0).astype(jnp.float32)
    z = z + is0 * w3s_ref[...] + bk_ref[...]
    msg_ref[0] = gate_ref[0] * jnp.maximum(z, 0.0)


def _msg2_body(hs_ref, hr_ref, cf_ref, gate_ref, src_ref, tgt_ref,
               w12_ref, w45_ref, w3s_ref, bk_ref, out_ref):
    g = pl.program_id(1)
    hs = hs_ref[0]
    hr = hr_ref[0]
    a = jnp.concatenate([hs * hr.astype(jnp.float32), hs], axis=1)
    b = jnp.concatenate([hr, cf_ref[0]], axis=1)
    z = (jnp.dot(a, w12_ref[...], preferred_element_type=jnp.float32)
         + jnp.dot(b, w45_ref[...], preferred_element_type=jnp.float32))
    is0 = (src_ref[0] == 0).astype(jnp.float32)
    z = z + is0 * w3s_ref[...] + bk_ref[...]
    wmsg = gate_ref[0] * jnp.maximum(z, 0.0)
    t0 = (tgt_ref[0] == 0).astype(jnp.float32)
    part = jnp.sum(wmsg * t0, axis=0, keepdims=True)

    @pl.when(g == 0)
    def _():
        out_ref[0] = part

    @pl.when(g > 0)
    def _():
        out_ref[0] = out_ref[0] + part


def _upd_body(ag_ref, h_ref, w_ref, ub_ref, lg_ref, lb_ref, out_ref):
    u = jnp.dot(ag_ref[0], w_ref[...],
                preferred_element_type=jnp.float32) + ub_ref[...]
    x = h_ref[0] + u
    m = jnp.mean(x, axis=1, keepdims=True)
    xc = x - m
    v = jnp.mean(xc * xc, axis=1, keepdims=True)
    out_ref[0] = xc * lax.rsqrt(v + 1e-5) * lg_ref[...] + lb_ref[...]


def _final_body(a0_ref, h0_ref, w_ref, ub_ref, lg_ref, lb_ref, out_ref):
    u = jnp.dot(a0_ref[:, 0, :], w_ref[...],
                preferred_element_type=jnp.float32) + ub_ref[...]
    x = h0_ref[:, 0, :] + u
    m = jnp.mean(x, axis=1, keepdims=True)
    xc = x - m
    v = jnp.mean(xc * xc, axis=1, keepdims=True)
    out_ref[:, 0, :] = xc * lax.rsqrt(v + 1e-5) * lg_ref[...] + lb_ref[...]


def kernel(edge_index, rels, scores, edge_conf_mask, edge_mask, mask,
           r_query_embed, conf_embeds, rel_embed_table, beta_w, beta_b,
           msg_w, msg_b, upd_w, upd_b, ln_g, ln_b):
    Bn, MaxN = mask.shape
    E = rels.shape[1]
    pe = E_PAD - E

    src = jnp.pad(edge_index[:, 0, :], ((0, 0), (0, pe)))
    tgt = jnp.pad(edge_index[:, 1, :], ((0, 0), (0, pe)))
    rels_f = jnp.pad(rels, ((0, 0), (0, pe))).reshape(-1)
    sc_c = jnp.pad(scores, ((0, 0), (0, pe)))[..., None]
    cm_c = jnp.pad(edge_conf_mask.astype(jnp.int32), ((0, 0), (0, pe)))[..., None]
    em_c = jnp.pad(edge_mask.astype(jnp.float32), ((0, 0), (0, pe)))[..., None]
    src_c = src[..., None]
    tgt_c = tgt[..., None]
    cf = jnp.pad(conf_embeds, ((0, 0), (0, pe), (0, 0))).astype(jnp.bfloat16)
    src_g = (src + jnp.arange(Bn, dtype=jnp.int32)[:, None] * N_PAD).reshape(-1)
    tgt_f = tgt.reshape(-1)
    rq = r_query_embed[:, None, :]
    bw_row = beta_w.reshape(1, D)
    bb = beta_b.reshape(1, 1)

    w1 = msg_w[:, 0:D, :]
    w2s = jnp.sum(msg_w[:, D:2 * D, :], axis=1, keepdims=True)
    w3s = jnp.sum(msg_w[:, 2 * D:3 * D, :], axis=1, keepdims=True)
    w14_0 = jnp.concatenate([w1[0], msg_w[0, 3 * D:4 * D, :]], axis=1)
    w5_0 = msg_w[0, 4 * D:5 * D, :].astype(jnp.bfloat16)
    w12 = jnp.concatenate([w1, msg_w[:, D:2 * D, :]], axis=1)
    w45bf = msg_w[:, 3 * D:5 * D, :].astype(jnp.bfloat16)
    mb = msg_b[:, None, :]
    ub = upd_b[:, None, :]
    lg = ln_g.reshape(1, D)
    lb = ln_b.reshape(1, D)

    h0 = jnp.zeros((Bn, N_PAD, D), jnp.float32).at[:, 0, :].set(1.0)

    feat, col, full2, grid = _edge_specs(Bn)
    node = pl.BlockSpec((1, 1024, D), lambda b, g: (b, g, 0))
    node_grid = (Bn, N_PAD // 1024)
    rq_spec = pl.BlockSpec((1, 1, D), lambda b, g: (b, 0, 0))

    n_tiles = NC * NS
    rep = jnp.broadcast_to(rel_embed_table[None], (n_tiles,) + rel_embed_table.shape)
    rep = rep.reshape(-1, D)
    rpt = rels_f.shape[0] // n_tiles
    rels_rep = rels_f + (jnp.arange(rels_f.shape[0], dtype=jnp.int32) // rpt) * N_RELS
    hr = _sc_gather_rows(rep, rels_rep).reshape(Bn, E_PAD, D)

    msg0, gate, hr_bf = pl.pallas_call(
        _msg0_body,
        grid=grid,
        in_specs=[feat, feat, col, col, col, col, rq_spec,
                  full2((1, D)), full2((1, 1)), full2((D, 2 * D)),
                  full2((D, D)), full2((1, D)), full2((1, D))],
        out_specs=[feat, col, feat],
        out_shape=[jax.ShapeDtypeStruct((Bn, E_PAD, D), jnp.float32),
                   jax.ShapeDtypeStruct((Bn, E_PAD, 1), jnp.float32),
                   jax.ShapeDtypeStruct((Bn, E_PAD, D), jnp.bfloat16)],
    )(hr, cf, sc_c, cm_c, em_c, src_c, rq, bw_row, bb,
      w14_0, w5_0, w2s[0] + w3s[0], mb[0])

    def run_update(aggr, h_prev, k):
        return pl.pallas_call(
            _upd_body,
            grid=node_grid,
            in_specs=[node, node, full2((D, D)), full2((1, D)),
                      full2((1, D)), full2((1, D))],
            out_specs=node,
            out_shape=jax.ShapeDtypeStruct((Bn, N_PAD, D), jnp.float32),
        )(aggr, h_prev, upd_w[k], ub[k], lg, lb)

    aggr0 = _sc_scatter_add(msg0.reshape(-1, D), tgt_f, Bn)
    h1 = run_update(aggr0, h0, 0)

    hs1 = _sc_gather_rows(h1.reshape(-1, D), src_g).reshape(Bn, E_PAD, D)
    msg1 = pl.pallas_call(
        _msgk_body,
        grid=grid,
        in_specs=[feat, feat, feat, col, col,
                  full2((2 * D, D)), full2((2 * D, D)), full2((1, D)),
                  full2((1, D))],
        out_specs=feat,
        out_shape=jax.ShapeDtypeStruct((Bn, E_PAD, D), jnp.float32),
    )(hs1, hr_bf, cf, gate, src_c, w12[1], w45bf[1], w3s[1], mb[1])
    aggr1 = _sc_scatter_add(msg1.reshape(-1, D), tgt_f, Bn)
    h2 = run_update(aggr1, h1, 1)

    hs2 = _sc_gather_rows(h2.reshape(-1, D), src_g).reshape(Bn, E_PAD, D)
    a0 = pl.pallas_call(
        _msg2_body,
        grid=grid,
        in_specs=[feat, feat, feat, col, col, col,
                  full2((2 * D, D)), full2((2 * D, D)), full2((1, D)),
                  full2((1, D))],
        out_specs=pl.BlockSpec((1, 1, D), lambda b, g: (b, 0, 0)),
        out_shape=jax.ShapeDtypeStruct((Bn, 1, D), jnp.float32),
    )(hs2, hr_bf, cf, gate, src_c, tgt_c, w12[2], w45bf[2], w3s[2], mb[2])

    full3 = lambda shape: pl.BlockSpec(shape, lambda: (0,) * len(shape))
    ctx2 = pl.pallas_call(
        _final_body,
        in_specs=[full3((Bn, 1, D)), full3((Bn, 1, D)), full3((D, D)),
                  full3((1, D)), full3((1, D)), full3((1, D))],
        out_specs=full3((Bn, 1, D)),
        out_shape=jax.ShapeDtypeStruct((Bn, 1, D), jnp.float32),
    )(a0, h2[:, :1, :], upd_w[2], ub[2], lg, lb)

    return jnp.stack([h1[:, 0, :], h2[:, 0, :], ctx2[:, 0, :]], axis=1)


# speedup vs baseline: 4.2371x; 1.0334x over previous
import functools

import jax
import jax.numpy as jnp
from jax import lax
from jax.experimental import pallas as pl
from jax.experimental.pallas import tpu as pltpu
from jax.experimental.pallas import tpu_sc as plsc

D = 128
N_RELS = 512
TAU = 0.1
E_PAD = 20480
N_PAD = 10240
NC = 2
NS = 16
CHUNK = 128
R = 512
RN = 1024
PB = 2


def _sc_gather_rows(table, idx, tok):
    M = idx.shape[0]
    dtype = table.dtype
    per_tile = M // (NC * NS)
    n_chunks = per_tile // CHUNK
    mesh = plsc.VectorSubcoreMesh(core_axis_name="c", subcore_axis_name="s")

    @functools.partial(
        pl.kernel,
        out_type=[jax.ShapeDtypeStruct((M, D), dtype),
                  jax.ShapeDtypeStruct((8,), jnp.int32)],
        mesh=mesh,
        scratch_types=[
            pltpu.VMEM((per_tile,), jnp.int32),
            pltpu.VMEM((CHUNK, D), dtype),
            pltpu.VMEM((CHUNK, D), dtype),
            pltpu.VMEM((8,), jnp.int32),
            pltpu.SemaphoreType.DMA,
            pltpu.SemaphoreType.DMA,
        ],
    )
    def k(table_hbm, idx_hbm, tok_hbm, out_hbm, tok_out_hbm,
          idx_all, rows0, rows1, tbuf, gsem, wsem):
        wid = lax.axis_index("s") * NC + lax.axis_index("c")
        base = wid * per_tile

        @pl.when(wid == 0)
        def _():
            pltpu.sync_copy(tok_hbm, tbuf)
            pltpu.sync_copy(tbuf, tok_out_hbm)
        pltpu.sync_copy(idx_hbm.at[pl.ds(base, per_tile)], idx_all)
        rows = (rows0, rows1)
        g = [None] * n_chunks
        w = [None] * n_chunks
        g[0] = pltpu.async_copy(
            table_hbm.at[idx_all.at[pl.ds(0, CHUNK)]], rows0, gsem)
        for j in range(n_chunks):
            g[j].wait()
            if j + 1 < n_chunks:
                if j >= 1:
                    w[j - 1].wait()
                g[j + 1] = pltpu.async_copy(
                    table_hbm.at[idx_all.at[pl.ds((j + 1) * CHUNK, CHUNK)]],
                    rows[(j + 1) % 2], gsem)
            w[j] = pltpu.async_copy(
                rows[j % 2], out_hbm.at[pl.ds(base + j * CHUNK, CHUNK)], wsem)
        if n_chunks >= 2:
            w[n_chunks - 2].wait()
        w[n_chunks - 1].wait()

    return k(table, idx, tok)


def _sc_scatter_add(msg, tgt, tok):
    per_tile = E_PAD // NS
    n_chunks = per_tile // CHUNK
    out_rows = N_PAD // NS
    n_out = out_rows // CHUNK
    ZR = CHUNK // 2
    mesh = plsc.VectorSubcoreMesh(core_axis_name="c", subcore_axis_name="s")

    @functools.partial(
        pl.kernel,
        out_type=[jax.ShapeDtypeStruct((PB, N_PAD, D), jnp.float32),
                  jax.ShapeDtypeStruct((8,), jnp.int32)],
        mesh=mesh,
        scratch_types=[
            pltpu.VMEM((CHUNK,), jnp.int32),
            pltpu.VMEM((CHUNK,), jnp.int32),
            pltpu.VMEM((CHUNK, D), jnp.float32),
            pltpu.VMEM((CHUNK, D), jnp.float32),
            pltpu.VMEM((CHUNK // 2, D), jnp.float32),
            pltpu.VMEM((8,), jnp.int32),
            pltpu.VMEM_SHARED((N_PAD, D), jnp.float32),
            pltpu.SemaphoreType.DMA,
            pltpu.SemaphoreType.DMA,
            pltpu.SemaphoreType.DMA,
        ],
    )
    def k(msg_hbm, tgt_hbm, tok_hbm, out_hbm, tok_out_hbm,
          ib0, ib1, mb0, mb1, zero_v, tbuf, aggr_sp, lsem, zsem, wsem):
        b = lax.axis_index("c")
        s = lax.axis_index("s")

        @pl.when((b == 0) & (s == 0))
        def _():
            pltpu.sync_copy(tok_hbm, tbuf)
            pltpu.sync_copy(tbuf, tok_out_hbm)
        ib = (ib0, ib1)
        mb = (mb0, mb1)
        ZR = CHUNK // 2
        ebase = b * E_PAD + s * per_tile
        nbase = s * out_rows

        def zrow(i, _):
            for t in range(D // 16):
                zero_v[i, pl.ds(t * 16, 16)] = jnp.zeros((16,), jnp.float32)
            return ()

        lax.fori_loop(0, ZR, zrow, ())

        zs = [pltpu.async_copy(
            zero_v, aggr_sp.at[pl.ds(nbase + j * ZR, ZR)], zsem)
            for j in range(2 * n_out)]
        for d in zs:
            d.wait()
        plsc.subcore_barrier()

        il = [None] * n_chunks
        ml = [None] * n_chunks
        il[0] = pltpu.async_copy(tgt_hbm.at[pl.ds(ebase, CHUNK)], ib0, lsem)
        ml[0] = pltpu.async_copy(msg_hbm.at[pl.ds(ebase, CHUNK)], mb0, lsem)
        for j in range(n_chunks):
            il[j].wait()
            ml[j].wait()
            if j + 1 < n_chunks:
                off = ebase + (j + 1) * CHUNK
                il[j + 1] = pltpu.async_copy(
                    tgt_hbm.at[pl.ds(off, CHUNK)], ib[(j + 1) % 2], lsem)
                ml[j + 1] = pltpu.async_copy(
                    msg_hbm.at[pl.ds(off, CHUNK)], mb[(j + 1) % 2], lsem)
            pltpu.sync_copy(mb[j % 2], aggr_sp.at[ib[j % 2]], add=True)
        plsc.subcore_barrier()

        wb = [None] * n_out
        for j in range(n_out):
            if j >= 2:
                wb[j - 2].wait()
            r0 = nbase + j * CHUNK
            pltpu.sync_copy(aggr_sp.at[pl.ds(r0, CHUNK)], mb[j % 2])
            wb[j] = pltpu.async_copy(
                mb[j % 2], out_hbm.at[b, pl.ds(r0, CHUNK)], wsem)
        wb[n_out - 2].wait()
        wb[n_out - 1].wait()

    return k(msg, tgt, tok)


def _msg0_body(hr_ref, cf_ref, sc_ref, cm_ref, em_ref, src_ref, rq_ref,
               bw_ref, bb_ref, w14_ref, w5_ref, v0_ref, b0_ref,
               msg_ref, gate_ref, hrbf_ref):
    hr = hr_ref[0]
    cf = cf_ref[0]
    hrbf_ref[0] = hr.astype(jnp.bfloat16)
    bwr = bw_ref[...]
    logit = (jnp.sum(hr * bwr, axis=1, keepdims=True)
             + jnp.sum(rq_ref[0] * bwr, axis=1, keepdims=True) + bb_ref[0, 0])
    beta = jax.nn.sigmoid(logit)
    gk = jax.nn.sigmoid((sc_ref[0] - beta) / TAU)
    gate = jnp.where(cm_ref[0] > 0, gk, 0.5) * em_ref[0]
    gate_ref[0] = gate
    is0 = (src_ref[0] == 0).astype(jnp.float32)
    t14 = jnp.dot(hr, w14_ref[...], preferred_element_type=jnp.float32)
    t5 = jnp.dot(cf, w5_ref[...], preferred_element_type=jnp.float32)
    z = is0 * (t14[:, :D] + v0_ref[...]) + t14[:, D:] + t5 + b0_ref[...]
    msg_ref[0] = gate * jnp.maximum(z, 0.0)


def _msgk_body(hs_ref, hr_ref, cf_ref, gate_ref, src_ref,
               w12_ref, w45_ref, w3s_ref, bk_ref, msg_ref):
    hs = hs_ref[0]
    hr = hr_ref[0]
    a = jnp.concatenate([hs * hr.astype(jnp.float32), hs], axis=1)
    bcat = jnp.concatenate([hr, cf_ref[0]], axis=1)
    z = (jnp.dot(a, w12_ref[...], preferred_element_type=jnp.float32)
         + jnp.dot(bcat, w45_ref[...], preferred_element_type=jnp.float32))
    is0 = (src_ref[0] == 0).astype(jnp.float32)
    z = z + is0 * w3s_ref[...] + bk_ref[...]
    msg_ref[0] = gate_ref[0] * jnp.maximum(z, 0.0)


def _msg2_body(hs_ref, hr_ref, cf_ref, gate_ref, src_ref, tgt_ref, h0_ref,
               w12_ref, w45_ref, w3s_ref, bk_ref, wu_ref, ub_ref,
               lg_ref, lb_ref, out_ref):
    g = pl.program_id(1)
    ng = pl.num_programs(1)
    hs = hs_ref[0]
    hr = hr_ref[0]
    a = jnp.concatenate([hs * hr.astype(jnp.float32), hs], axis=1)
    bcat = jnp.concatenate([hr, cf_ref[0]], axis=1)
    z = (jnp.dot(a, w12_ref[...], preferred_element_type=jnp.float32)
         + jnp.dot(bcat, w45_ref[...], preferred_element_type=jnp.float32))
    is0 = (src_ref[0] == 0).astype(jnp.float32)
    z = z + is0 * w3s_ref[...] + bk_ref[...]
    wmsg = gate_ref[0] * jnp.maximum(z, 0.0)
    t0 = (tgt_ref[0] == 0).astype(jnp.float32)
    part = jnp.sum(wmsg * t0, axis=0, keepdims=True)

    @pl.when(g == 0)
    def _():
        out_ref[0] = part

    @pl.when((g > 0) & (g < ng - 1))
    def _():
        out_ref[0] = out_ref[0] + part

    @pl.when(g == ng - 1)
    def _():
        aggr0 = out_ref[0] + part
        u = jnp.dot(aggr0, wu_ref[...],
                    preferred_element_type=jnp.float32) + ub_ref[...]
        x = h0_ref[0] + u
        m = jnp.mean(x, axis=1, keepdims=True)
        xc = x - m
        v = jnp.mean(xc * xc, axis=1, keepdims=True)
        out_ref[0] = xc * lax.rsqrt(v + 1e-5) * lg_ref[...] + lb_ref[...]


def _upd0_body(ag_ref, w_ref, ub_ref, lg_ref, lb_ref, out_ref):
    g = pl.program_id(1)
    u = jnp.dot(ag_ref[0], w_ref[...],
                preferred_element_type=jnp.float32) + ub_ref[...]
    rows = lax.broadcasted_iota(jnp.int32, (RN, D), 0)
    ind = ((rows == 0) & (g == 0)).astype(jnp.float32)
    x = ind + u
    m = jnp.mean(x, axis=1, keepdims=True)
    xc = x - m
    v = jnp.mean(xc * xc, axis=1, keepdims=True)
    out_ref[0] = xc * lax.rsqrt(v + 1e-5) * lg_ref[...] + lb_ref[...]


def _upd_body(ag_ref, h_ref, w_ref, ub_ref, lg_ref, lb_ref, out_ref):
    u = jnp.dot(ag_ref[0], w_ref[...],
                preferred_element_type=jnp.float32) + ub_ref[...]
    x = h_ref[0] + u
    m = jnp.mean(x, axis=1, keepdims=True)
    xc = x - m
    v = jnp.mean(xc * xc, axis=1, keepdims=True)
    out_ref[0] = xc * lax.rsqrt(v + 1e-5) * lg_ref[...] + lb_ref[...]


def kernel(edge_index, rels, scores, edge_conf_mask, edge_mask, mask,
           r_query_embed, conf_embeds, rel_embed_table, beta_w, beta_b,
           msg_w, msg_b, upd_w, upd_b, ln_g, ln_b):
    Bn, MaxN = mask.shape
    E = rels.shape[1]
    pe = E_PAD - E
    n_tiles = NC * NS

    src = jnp.pad(edge_index[:, 0, :], ((0, 0), (0, pe)))
    tgt = jnp.pad(edge_index[:, 1, :], ((0, 0), (0, pe)))
    rels_p = jnp.pad(rels, ((0, 0), (0, pe)))
    sc_c = jnp.pad(scores, ((0, 0), (0, pe)))[..., None]
    cm_c = jnp.pad(edge_conf_mask.astype(jnp.int32), ((0, 0), (0, pe)))[..., None]
    em_c = jnp.pad(edge_mask.astype(jnp.float32), ((0, 0), (0, pe)))[..., None]
    src_c = src[..., None]
    tgt_c = tgt[..., None]
    cf = jnp.pad(conf_embeds, ((0, 0), (0, pe), (0, 0))).astype(jnp.bfloat16)
    rq = r_query_embed[:, None, :]
    bw_row = beta_w.reshape(1, D)
    bb = beta_b.reshape(1, 1)

    rep = jnp.broadcast_to(
        rel_embed_table[None], (n_tiles,) + rel_embed_table.shape)
    rep = rep.reshape(-1, D)
    M = PB * E_PAD
    rpt = M // n_tiles
    tile_off = (jnp.arange(M, dtype=jnp.int32) // rpt) * N_RELS
    boff = jnp.arange(PB, dtype=jnp.int32)[:, None] * N_PAD

    w1 = msg_w[:, 0:D, :]
    w2s = jnp.sum(msg_w[:, D:2 * D, :], axis=1, keepdims=True)
    w3s = jnp.sum(msg_w[:, 2 * D:3 * D, :], axis=1, keepdims=True)
    w14_0 = jnp.concatenate([w1[0], msg_w[0, 3 * D:4 * D, :]], axis=1)
    w5_0 = msg_w[0, 4 * D:5 * D, :].astype(jnp.bfloat16)
    w12 = jnp.concatenate([w1, msg_w[:, D:2 * D, :]], axis=1)
    w45bf = msg_w[:, 3 * D:5 * D, :].astype(jnp.bfloat16)
    mb = msg_b[:, None, :]
    ub = upd_b[:, None, :]
    lg = ln_g.reshape(1, D)
    lb = ln_b.reshape(1, D)

    feat = pl.BlockSpec((1, R, D), lambda b, g: (b, g, 0))
    col = pl.BlockSpec((1, R, 1), lambda b, g: (b, g, 0))
    full2 = lambda shape: pl.BlockSpec(shape, lambda b, g: (0, 0))
    grid = (PB, E_PAD // R)
    node = pl.BlockSpec((1, RN, D), lambda b, g: (b, g, 0))
    node_grid = (PB, N_PAD // RN)
    rq_spec = pl.BlockSpec((1, 1, D), lambda b, g: (b, 0, 0))
    acc_spec = pl.BlockSpec((1, 1, D), lambda b, g: (b, 0, 0))

    tok = jnp.zeros((8,), jnp.int32)
    npair = Bn // PB
    rels_f = [None] * npair
    src_g = [None] * npair
    tgt_f = [None] * npair
    sls = [slice(q * PB, q * PB + PB) for q in range(npair)]
    for q in range(npair):
        rels_f[q] = rels_p[sls[q]].reshape(-1) + tile_off
        src_g[q] = (src[sls[q]] + boff).reshape(-1)
        tgt_f[q] = tgt[sls[q]].reshape(-1)

    hr = [None] * npair
    for q in range(npair):
        hr[q], tok = _sc_gather_rows(rep, rels_f[q], tok)
        hr[q] = hr[q].reshape(PB, E_PAD, D)

    msg0, gate, hr_bf = [None] * npair, [None] * npair, [None] * npair
    for q in range(npair):
        msg0[q], gate[q], hr_bf[q] = pl.pallas_call(
            _msg0_body,
            grid=grid,
            in_specs=[feat, feat, col, col, col, col, rq_spec,
                      full2((1, D)), full2((1, 1)), full2((D, 2 * D)),
                      full2((D, D)), full2((1, D)), full2((1, D))],
            out_specs=[feat, col, feat],
            out_shape=[jax.ShapeDtypeStruct((PB, E_PAD, D), jnp.float32),
                       jax.ShapeDtypeStruct((PB, E_PAD, 1), jnp.float32),
                       jax.ShapeDtypeStruct((PB, E_PAD, D), jnp.bfloat16)],
        )(hr[q], cf[sls[q]], sc_c[sls[q]], cm_c[sls[q]], em_c[sls[q]],
          src_c[sls[q]], rq[sls[q]], bw_row, bb, w14_0, w5_0,
          w2s[0] + w3s[0], mb[0])

    aggr0 = [None] * npair
    for q in range(npair):
        aggr0[q], tok = _sc_scatter_add(msg0[q].reshape(-1, D), tgt_f[q], tok)

    h1 = [None] * npair
    for q in range(npair):
        h1[q] = pl.pallas_call(
            _upd0_body,
            grid=node_grid,
            in_specs=[node, full2((D, D)), full2((1, D)),
                      full2((1, D)), full2((1, D))],
            out_specs=node,
            out_shape=jax.ShapeDtypeStruct((PB, N_PAD, D), jnp.float32),
        )(aggr0[q], upd_w[0], ub[0], lg, lb)

    hs1 = [None] * npair
    for q in range(npair):
        hs1[q], tok = _sc_gather_rows(h1[q].reshape(-1, D), src_g[q], tok)
        hs1[q] = hs1[q].reshape(PB, E_PAD, D)

    msg1 = [None] * npair
    for q in range(npair):
        msg1[q] = pl.pallas_call(
            _msgk_body,
            grid=grid,
            in_specs=[feat, feat, feat, col, col,
                      full2((2 * D, D)), full2((2 * D, D)), full2((1, D)),
                      full2((1, D))],
            out_specs=feat,
            out_shape=jax.ShapeDtypeStruct((PB, E_PAD, D), jnp.float32),
        )(hs1[q], hr_bf[q], cf[sls[q]], gate[q], src_c[sls[q]],
          w12[1], w45bf[1], w3s[1], mb[1])

    aggr1 = [None] * npair
    for q in range(npair):
        aggr1[q], tok = _sc_scatter_add(msg1[q].reshape(-1, D), tgt_f[q], tok)

    h2 = [None] * npair
    for q in range(npair):
        h2[q] = pl.pallas_call(
            _upd_body,
            grid=node_grid,
            in_specs=[node, node, full2((D, D)), full2((1, D)),
                      full2((1, D)), full2((1, D))],
            out_specs=node,
            out_shape=jax.ShapeDtypeStruct((PB, N_PAD, D), jnp.float32),
        )(aggr1[q], h1[q], upd_w[1], ub[1], lg, lb)

    hs2 = [None] * npair
    for q in range(npair):
        hs2[q], tok = _sc_gather_rows(h2[q].reshape(-1, D), src_g[q], tok)
        hs2[q] = hs2[q].reshape(PB, E_PAD, D)

    ctx = []
    for q in range(npair):
        ctx2 = pl.pallas_call(
            _msg2_body,
            grid=grid,
            in_specs=[feat, feat, feat, col, col, col, acc_spec,
                      full2((2 * D, D)), full2((2 * D, D)), full2((1, D)),
                      full2((1, D)), full2((D, D)), full2((1, D)),
                      full2((1, D)), full2((1, D))],
            out_specs=acc_spec,
            out_shape=jax.ShapeDtypeStruct((PB, 1, D), jnp.float32),
        )(hs2[q], hr_bf[q], cf[sls[q]], gate[q], src_c[sls[q]],
          tgt_c[sls[q]], h2[q][:, :1, :],
          w12[2], w45bf[2], w3s[2], mb[2], upd_w[2], ub[2], lg, lb)
        ctx.append(jnp.stack(
            [h1[q][:, 0, :], h2[q][:, 0, :], ctx2[:, 0, :]], axis=1))

    return jnp.concatenate(ctx, axis=0)


# speedup vs baseline: 4.3325x; 1.0225x over previous
import functools

import jax
import jax.numpy as jnp
from jax import lax
from jax.experimental import pallas as pl
from jax.experimental.pallas import tpu as pltpu
from jax.experimental.pallas import tpu_sc as plsc

D = 128
N_RELS = 512
TAU = 0.1
E_PAD = 20480
N_PAD = 10240
NC = 2
NS = 16
CHUNK = 128
R = 512
RN = 1024
PB = 2


def _sc_gather_rows(table, idx, tok):
    M = idx.shape[0]
    dtype = table.dtype
    per_tile = M // (NC * NS)
    n_chunks = per_tile // CHUNK
    mesh = plsc.VectorSubcoreMesh(core_axis_name="c", subcore_axis_name="s")

    @functools.partial(
        pl.kernel,
        out_type=[jax.ShapeDtypeStruct((M, D), dtype),
                  jax.ShapeDtypeStruct((8,), jnp.int32)],
        mesh=mesh,
        scratch_types=[
            pltpu.VMEM((per_tile,), jnp.int32),
            pltpu.VMEM((CHUNK, D), dtype),
            pltpu.VMEM((CHUNK, D), dtype),
            pltpu.VMEM((CHUNK, D), dtype),
            pltpu.VMEM((CHUNK, D), dtype),
            pltpu.VMEM((8,), jnp.int32),
            pltpu.SemaphoreType.DMA,
            pltpu.SemaphoreType.DMA,
        ],
    )
    def k(table_hbm, idx_hbm, tok_hbm, out_hbm, tok_out_hbm,
          idx_all, rows0, rows1, rows2, rows3, tbuf, gsem, wsem):
        wid = lax.axis_index("s") * NC + lax.axis_index("c")
        base = wid * per_tile

        @pl.when(wid == 0)
        def _():
            pltpu.sync_copy(tok_hbm, tbuf)
            pltpu.sync_copy(tbuf, tok_out_hbm)
        pltpu.sync_copy(idx_hbm.at[pl.ds(base, per_tile)], idx_all)
        rows = (rows0, rows1, rows2, rows3)
        nd = len(rows)
        g = [None] * n_chunks
        w = [None] * n_chunks

        def fire(j):
            return pltpu.async_copy(
                table_hbm.at[idx_all.at[pl.ds(j * CHUNK, CHUNK)]],
                rows[j % nd], gsem)

        for j in range(min(nd - 1, n_chunks)):
            g[j] = fire(j)
        for j in range(n_chunks):
            g[j].wait()
            nxt = j + nd - 1
            if nxt < n_chunks:
                if j >= 1:
                    w[j - 1].wait()
                g[nxt] = fire(nxt)
            w[j] = pltpu.async_copy(
                rows[j % nd], out_hbm.at[pl.ds(base + j * CHUNK, CHUNK)], wsem)
        for j in range(max(0, n_chunks - (nd - 1)), n_chunks):
            w[j].wait()

    return k(table, idx, tok)


def _sc_scatter_add(msg, tgt, tok):
    per_tile = E_PAD // NS
    n_chunks = per_tile // CHUNK
    out_rows = N_PAD // NS
    n_out = out_rows // CHUNK
    ZR = CHUNK // 2
    mesh = plsc.VectorSubcoreMesh(core_axis_name="c", subcore_axis_name="s")

    @functools.partial(
        pl.kernel,
        out_type=[jax.ShapeDtypeStruct((PB, N_PAD, D), jnp.float32),
                  jax.ShapeDtypeStruct((8,), jnp.int32)],
        mesh=mesh,
        scratch_types=[
            pltpu.VMEM((CHUNK,), jnp.int32),
            pltpu.VMEM((CHUNK,), jnp.int32),
            pltpu.VMEM((CHUNK, D), jnp.float32),
            pltpu.VMEM((CHUNK, D), jnp.float32),
            pltpu.VMEM((CHUNK // 2, D), jnp.float32),
            pltpu.VMEM((8,), jnp.int32),
            pltpu.VMEM_SHARED((N_PAD, D), jnp.float32),
            pltpu.SemaphoreType.DMA,
            pltpu.SemaphoreType.DMA,
            pltpu.SemaphoreType.DMA,
        ],
    )
    def k(msg_hbm, tgt_hbm, tok_hbm, out_hbm, tok_out_hbm,
          ib0, ib1, mb0, mb1, zero_v, tbuf, aggr_sp, lsem, zsem, wsem):
        b = lax.axis_index("c")
        s = lax.axis_index("s")

        @pl.when((b == 0) & (s == 0))
        def _():
            pltpu.sync_copy(tok_hbm, tbuf)
            pltpu.sync_copy(tbuf, tok_out_hbm)
        ib = (ib0, ib1)
        mb = (mb0, mb1)
        ZR = CHUNK // 2
        ebase = b * E_PAD + s * per_tile
        nbase = s * out_rows

        def zrow(i, _):
            for t in range(D // 16):
                zero_v[i, pl.ds(t * 16, 16)] = jnp.zeros((16,), jnp.float32)
            return ()

        lax.fori_loop(0, ZR, zrow, ())

        zs = [pltpu.async_copy(
            zero_v, aggr_sp.at[pl.ds(nbase + j * ZR, ZR)], zsem)
            for j in range(2 * n_out)]
        for d in zs:
            d.wait()
        plsc.subcore_barrier()

        il = [None] * n_chunks
        ml = [None] * n_chunks
        il[0] = pltpu.async_copy(tgt_hbm.at[pl.ds(ebase, CHUNK)], ib0, lsem)
        ml[0] = pltpu.async_copy(msg_hbm.at[pl.ds(ebase, CHUNK)], mb0, lsem)
        for j in range(n_chunks):
            il[j].wait()
            ml[j].wait()
            if j + 1 < n_chunks:
                off = ebase + (j + 1) * CHUNK
                il[j + 1] = pltpu.async_copy(
                    tgt_hbm.at[pl.ds(off, CHUNK)], ib[(j + 1) % 2], lsem)
                ml[j + 1] = pltpu.async_copy(
                    msg_hbm.at[pl.ds(off, CHUNK)], mb[(j + 1) % 2], lsem)
            pltpu.sync_copy(mb[j % 2], aggr_sp.at[ib[j % 2]], add=True)
        plsc.subcore_barrier()

        wb = [None] * n_out
        for j in range(n_out):
            if j >= 2:
                wb[j - 2].wait()
            r0 = nbase + j * CHUNK
            pltpu.sync_copy(aggr_sp.at[pl.ds(r0, CHUNK)], mb[j % 2])
            wb[j] = pltpu.async_copy(
                mb[j % 2], out_hbm.at[b, pl.ds(r0, CHUNK)], wsem)
        wb[n_out - 2].wait()
        wb[n_out - 1].wait()

    return k(msg, tgt, tok)


def _msg0_body(hr_ref, cf_ref, sc_ref, cm_ref, em_ref, src_ref, rq_ref,
               bw_ref, bb_ref, w14_ref, w5_ref, v0_ref, b0_ref,
               msg_ref, gate_ref, hrbf_ref):
    hr = hr_ref[0]
    cf = cf_ref[0]
    hrbf_ref[0] = hr.astype(jnp.bfloat16)
    bwr = bw_ref[...]
    logit = (jnp.sum(hr * bwr, axis=1, keepdims=True)
             + jnp.sum(rq_ref[0] * bwr, axis=1, keepdims=True) + bb_ref[0, 0])
    beta = jax.nn.sigmoid(logit)
    gk = jax.nn.sigmoid((sc_ref[0] - beta) / TAU)
    gate = jnp.where(cm_ref[0] > 0, gk, 0.5) * em_ref[0]
    gate_ref[0] = gate
    is0 = (src_ref[0] == 0).astype(jnp.float32)
    t14 = jnp.dot(hr, w14_ref[...], preferred_element_type=jnp.float32)
    t5 = jnp.dot(cf, w5_ref[...], preferred_element_type=jnp.float32)
    z = is0 * (t14[:, :D] + v0_ref[...]) + t14[:, D:] + t5 + b0_ref[...]
    msg_ref[0] = gate * jnp.maximum(z, 0.0)


def _msgk_body(hs_ref, hr_ref, cf_ref, gate_ref, src_ref,
               w12_ref, w45_ref, w3s_ref, bk_ref, msg_ref):
    hs = hs_ref[0]
    hr = hr_ref[0]
    a = jnp.concatenate([hs * hr.astype(jnp.float32), hs], axis=1)
    bcat = jnp.concatenate([hr, cf_ref[0]], axis=1)
    z = (jnp.dot(a, w12_ref[...], preferred_element_type=jnp.float32)
         + jnp.dot(bcat, w45_ref[...], preferred_element_type=jnp.float32))
    is0 = (src_ref[0] == 0).astype(jnp.float32)
    z = z + is0 * w3s_ref[...] + bk_ref[...]
    msg_ref[0] = gate_ref[0] * jnp.maximum(z, 0.0)


def _msg2_body(hs_ref, hr_ref, cf_ref, gate_ref, src_ref, tgt_ref, h0_ref,
               w12_ref, w45_ref, w3s_ref, bk_ref, wu_ref, ub_ref,
               lg_ref, lb_ref, out_ref):
    g = pl.program_id(1)
    ng = pl.num_programs(1)
    hs = hs_ref[0]
    hr = hr_ref[0]
    a = jnp.concatenate([hs * hr.astype(jnp.float32), hs], axis=1)
    bcat = jnp.concatenate([hr, cf_ref[0]], axis=1)
    z = (jnp.dot(a, w12_ref[...], preferred_element_type=jnp.float32)
         + jnp.dot(bcat, w45_ref[...], preferred_element_type=jnp.float32))
    is0 = (src_ref[0] == 0).astype(jnp.float32)
    z = z + is0 * w3s_ref[...] + bk_ref[...]
    wmsg = gate_ref[0] * jnp.maximum(z, 0.0)
    t0 = (tgt_ref[0] == 0).astype(jnp.float32)
    part = jnp.sum(wmsg * t0, axis=0, keepdims=True)

    @pl.when(g == 0)
    def _():
        out_ref[0] = part

    @pl.when((g > 0) & (g < ng - 1))
    def _():
        out_ref[0] = out_ref[0] + part

    @pl.when(g == ng - 1)
    def _():
        aggr0 = out_ref[0] + part
        u = jnp.dot(aggr0, wu_ref[...],
                    preferred_element_type=jnp.float32) + ub_ref[...]
        x = h0_ref[0] + u
        m = jnp.mean(x, axis=1, keepdims=True)
        xc = x - m
        v = jnp.mean(xc * xc, axis=1, keepdims=True)
        out_ref[0] = xc * lax.rsqrt(v + 1e-5) * lg_ref[...] + lb_ref[...]


def _upd0_body(ag_ref, w_ref, ub_ref, lg_ref, lb_ref, out_ref):
    g = pl.program_id(1)
    u = jnp.dot(ag_ref[0], w_ref[...],
                preferred_element_type=jnp.float32) + ub_ref[...]
    rows = lax.broadcasted_iota(jnp.int32, (RN, D), 0)
    ind = ((rows == 0) & (g == 0)).astype(jnp.float32)
    x = ind + u
    m = jnp.mean(x, axis=1, keepdims=True)
    xc = x - m
    v = jnp.mean(xc * xc, axis=1, keepdims=True)
    out_ref[0] = xc * lax.rsqrt(v + 1e-5) * lg_ref[...] + lb_ref[...]


def _upd_body(ag_ref, h_ref, w_ref, ub_ref, lg_ref, lb_ref, out_ref):
    u = jnp.dot(ag_ref[0], w_ref[...],
                preferred_element_type=jnp.float32) + ub_ref[...]
    x = h_ref[0] + u
    m = jnp.mean(x, axis=1, keepdims=True)
    xc = x - m
    v = jnp.mean(xc * xc, axis=1, keepdims=True)
    out_ref[0] = xc * lax.rsqrt(v + 1e-5) * lg_ref[...] + lb_ref[...]


def kernel(edge_index, rels, scores, edge_conf_mask, edge_mask, mask,
           r_query_embed, conf_embeds, rel_embed_table, beta_w, beta_b,
           msg_w, msg_b, upd_w, upd_b, ln_g, ln_b):
    Bn, MaxN = mask.shape
    E = rels.shape[1]
    pe = E_PAD - E
    n_tiles = NC * NS

    src = jnp.pad(edge_index[:, 0, :], ((0, 0), (0, pe)))
    tgt = jnp.pad(edge_index[:, 1, :], ((0, 0), (0, pe)))
    rels_p = jnp.pad(rels, ((0, 0), (0, pe)))
    sc_c = jnp.pad(scores, ((0, 0), (0, pe)))[..., None]
    cm_c = jnp.pad(edge_conf_mask.astype(jnp.int32), ((0, 0), (0, pe)))[..., None]
    em_c = jnp.pad(edge_mask.astype(jnp.float32), ((0, 0), (0, pe)))[..., None]
    src_c = src[..., None]
    tgt_c = tgt[..., None]
    cf = jnp.pad(conf_embeds, ((0, 0), (0, pe), (0, 0))).astype(jnp.bfloat16)
    rq = r_query_embed[:, None, :]
    bw_row = beta_w.reshape(1, D)
    bb = beta_b.reshape(1, 1)

    rep = jnp.broadcast_to(
        rel_embed_table[None], (n_tiles,) + rel_embed_table.shape)
    rep = rep.reshape(-1, D)
    MA = Bn * E_PAD
    rpt = MA // n_tiles
    tile_off = (jnp.arange(MA, dtype=jnp.int32) // rpt) * N_RELS
    boff = jnp.arange(PB, dtype=jnp.int32)[:, None] * N_PAD

    w1 = msg_w[:, 0:D, :]
    w2s = jnp.sum(msg_w[:, D:2 * D, :], axis=1, keepdims=True)
    w3s = jnp.sum(msg_w[:, 2 * D:3 * D, :], axis=1, keepdims=True)
    w14_0 = jnp.concatenate([w1[0], msg_w[0, 3 * D:4 * D, :]], axis=1)
    w5_0 = msg_w[0, 4 * D:5 * D, :].astype(jnp.bfloat16)
    w12 = jnp.concatenate([w1, msg_w[:, D:2 * D, :]], axis=1)
    w45bf = msg_w[:, 3 * D:5 * D, :].astype(jnp.bfloat16)
    mb = msg_b[:, None, :]
    ub = upd_b[:, None, :]
    lg = ln_g.reshape(1, D)
    lb = ln_b.reshape(1, D)

    feat = pl.BlockSpec((1, R, D), lambda b, g: (b, g, 0))
    col = pl.BlockSpec((1, R, 1), lambda b, g: (b, g, 0))
    full2 = lambda shape: pl.BlockSpec(shape, lambda b, g: (0, 0))
    grid = (PB, E_PAD // R)
    node = pl.BlockSpec((1, RN, D), lambda b, g: (b, g, 0))
    node_grid = (PB, N_PAD // RN)
    rq_spec = pl.BlockSpec((1, 1, D), lambda b, g: (b, 0, 0))
    acc_spec = pl.BlockSpec((1, 1, D), lambda b, g: (b, 0, 0))

    tok = jnp.zeros((8,), jnp.int32)
    npair = Bn // PB
    rels_f = [None] * npair
    src_g = [None] * npair
    tgt_f = [None] * npair
    sls = [slice(q * PB, q * PB + PB) for q in range(npair)]
    for q in range(npair):
        src_g[q] = (src[sls[q]] + boff).reshape(-1)
        tgt_f[q] = tgt[sls[q]].reshape(-1)

    rels_all = rels_p.reshape(-1) + tile_off
    hr_all, tok = _sc_gather_rows(rep, rels_all, tok)
    hr_all = hr_all.reshape(Bn, E_PAD, D)
    hr = [hr_all[sls[q]] for q in range(npair)]

    msg0, gate, hr_bf = [None] * npair, [None] * npair, [None] * npair
    for q in range(npair):
        msg0[q], gate[q], hr_bf[q] = pl.pallas_call(
            _msg0_body,
            grid=grid,
            in_specs=[feat, feat, col, col, col, col, rq_spec,
                      full2((1, D)), full2((1, 1)), full2((D, 2 * D)),
                      full2((D, D)), full2((1, D)), full2((1, D))],
            out_specs=[feat, col, feat],
            out_shape=[jax.ShapeDtypeStruct((PB, E_PAD, D), jnp.float32),
                       jax.ShapeDtypeStruct((PB, E_PAD, 1), jnp.float32),
                       jax.ShapeDtypeStruct((PB, E_PAD, D), jnp.bfloat16)],
        )(hr[q], cf[sls[q]], sc_c[sls[q]], cm_c[sls[q]], em_c[sls[q]],
          src_c[sls[q]], rq[sls[q]], bw_row, bb, w14_0, w5_0,
          w2s[0] + w3s[0], mb[0])

    aggr0 = [None] * npair
    for q in range(npair):
        aggr0[q], tok = _sc_scatter_add(msg0[q].reshape(-1, D), tgt_f[q], tok)

    h1 = [None] * npair
    for q in range(npair):
        h1[q] = pl.pallas_call(
            _upd0_body,
            grid=node_grid,
            in_specs=[node, full2((D, D)), full2((1, D)),
                      full2((1, D)), full2((1, D))],
            out_specs=node,
            out_shape=jax.ShapeDtypeStruct((PB, N_PAD, D), jnp.float32),
        )(aggr0[q], upd_w[0], ub[0], lg, lb)

    hs1 = [None] * npair
    for q in range(npair):
        hs1[q], tok = _sc_gather_rows(h1[q].reshape(-1, D), src_g[q], tok)
        hs1[q] = hs1[q].reshape(PB, E_PAD, D)

    msg1 = [None] * npair
    for q in range(npair):
        msg1[q] = pl.pallas_call(
            _msgk_body,
            grid=grid,
            in_specs=[feat, feat, feat, col, col,
                      full2((2 * D, D)), full2((2 * D, D)), full2((1, D)),
                      full2((1, D))],
            out_specs=feat,
            out_shape=jax.ShapeDtypeStruct((PB, E_PAD, D), jnp.float32),
        )(hs1[q], hr_bf[q], cf[sls[q]], gate[q], src_c[sls[q]],
          w12[1], w45bf[1], w3s[1], mb[1])

    aggr1 = [None] * npair
    for q in range(npair):
        aggr1[q], tok = _sc_scatter_add(msg1[q].reshape(-1, D), tgt_f[q], tok)

    h2 = [None] * npair
    for q in range(npair):
        h2[q] = pl.pallas_call(
            _upd_body,
            grid=node_grid,
            in_specs=[node, node, full2((D, D)), full2((1, D)),
                      full2((1, D)), full2((1, D))],
            out_specs=node,
            out_shape=jax.ShapeDtypeStruct((PB, N_PAD, D), jnp.float32),
        )(aggr1[q], h1[q], upd_w[1], ub[1], lg, lb)

    hs2 = [None] * npair
    for q in range(npair):
        hs2[q], tok = _sc_gather_rows(h2[q].reshape(-1, D), src_g[q], tok)
        hs2[q] = hs2[q].reshape(PB, E_PAD, D)

    ctx = []
    for q in range(npair):
        ctx2 = pl.pallas_call(
            _msg2_body,
            grid=grid,
            in_specs=[feat, feat, feat, col, col, col, acc_spec,
                      full2((2 * D, D)), full2((2 * D, D)), full2((1, D)),
                      full2((1, D)), full2((D, D)), full2((1, D)),
                      full2((1, D)), full2((1, D))],
            out_specs=acc_spec,
            out_shape=jax.ShapeDtypeStruct((PB, 1, D), jnp.float32),
        )(hs2[q], hr_bf[q], cf[sls[q]], gate[q], src_c[sls[q]],
          tgt_c[sls[q]], h2[q][:, :1, :],
          w12[2], w45bf[2], w3s[2], mb[2], upd_w[2], ub[2], lg, lb)
        ctx.append(jnp.stack(
            [h1[q][:, 0, :], h2[q][:, 0, :], ctx2[:, 0, :]], axis=1))

    return jnp.concatenate(ctx, axis=0)


# speedup vs baseline: 4.3542x; 1.0050x over previous
import functools

import jax
import jax.numpy as jnp
from jax import lax
from jax.experimental import pallas as pl
from jax.experimental.pallas import tpu as pltpu
from jax.experimental.pallas import tpu_sc as plsc

D = 128
N_RELS = 512
TAU = 0.1
E_PAD = 20480
N_PAD = 10240
NC = 2
NS = 16
CHUNK = 128
R = 512
RN = 1024
PB = 2


def _sc_gather_rows(table, idx, tok):
    M = idx.shape[0]
    dtype = table.dtype
    per_tile = M // (NC * NS)
    n_chunks = per_tile // CHUNK
    mesh = plsc.VectorSubcoreMesh(core_axis_name="c", subcore_axis_name="s")

    @functools.partial(
        pl.kernel,
        out_type=[jax.ShapeDtypeStruct((M, D), dtype),
                  jax.ShapeDtypeStruct((8,), jnp.int32)],
        mesh=mesh,
        scratch_types=[
            pltpu.VMEM((per_tile,), jnp.int32),
            pltpu.VMEM((CHUNK, D), dtype),
            pltpu.VMEM((CHUNK, D), dtype),
            pltpu.VMEM((CHUNK, D), dtype),
            pltpu.VMEM((CHUNK, D), dtype),
            pltpu.VMEM((8,), jnp.int32),
            pltpu.SemaphoreType.DMA,
            pltpu.SemaphoreType.DMA,
        ],
    )
    def k(table_hbm, idx_hbm, tok_hbm, out_hbm, tok_out_hbm,
          idx_all, rows0, rows1, rows2, rows3, tbuf, gsem, wsem):
        wid = lax.axis_index("s") * NC + lax.axis_index("c")
        base = wid * per_tile

        @pl.when(wid == 0)
        def _():
            pltpu.sync_copy(tok_hbm, tbuf)
            pltpu.sync_copy(tbuf, tok_out_hbm)
        pltpu.sync_copy(idx_hbm.at[pl.ds(base, per_tile)], idx_all)
        rows = (rows0, rows1, rows2, rows3)
        nd = len(rows)
        g = [None] * n_chunks
        w = [None] * n_chunks

        def fire(j):
            return pltpu.async_copy(
                table_hbm.at[idx_all.at[pl.ds(j * CHUNK, CHUNK)]],
                rows[j % nd], gsem)

        for j in range(min(nd - 1, n_chunks)):
            g[j] = fire(j)
        for j in range(n_chunks):
            g[j].wait()
            nxt = j + nd - 1
            if nxt < n_chunks:
                if j >= 1:
                    w[j - 1].wait()
                g[nxt] = fire(nxt)
            w[j] = pltpu.async_copy(
                rows[j % nd], out_hbm.at[pl.ds(base + j * CHUNK, CHUNK)], wsem)
        for j in range(max(0, n_chunks - (nd - 1)), n_chunks):
            w[j].wait()

    return k(table, idx, tok)


def _sc_scatter_add(msg, tgt, tok):
    per_tile = E_PAD // NS
    n_chunks = per_tile // CHUNK
    out_rows = N_PAD // NS
    n_out = out_rows // CHUNK
    ZR = CHUNK // 2
    mesh = plsc.VectorSubcoreMesh(core_axis_name="c", subcore_axis_name="s")

    @functools.partial(
        pl.kernel,
        out_type=[jax.ShapeDtypeStruct((PB, N_PAD, D), jnp.float32),
                  jax.ShapeDtypeStruct((8,), jnp.int32)],
        mesh=mesh,
        scratch_types=[
            pltpu.VMEM((CHUNK,), jnp.int32),
            pltpu.VMEM((CHUNK,), jnp.int32),
            pltpu.VMEM((CHUNK, D), jnp.float32),
            pltpu.VMEM((CHUNK, D), jnp.float32),
            pltpu.VMEM((CHUNK // 2, D), jnp.float32),
            pltpu.VMEM((8,), jnp.int32),
            pltpu.VMEM_SHARED((N_PAD, D), jnp.float32),
            pltpu.SemaphoreType.DMA,
            pltpu.SemaphoreType.DMA,
            pltpu.SemaphoreType.DMA,
        ],
    )
    def k(msg_hbm, tgt_hbm, tok_hbm, out_hbm, tok_out_hbm,
          ib0, ib1, mb0, mb1, zero_v, tbuf, aggr_sp, lsem, zsem, wsem):
        b = lax.axis_index("c")
        s = lax.axis_index("s")

        @pl.when((b == 0) & (s == 0))
        def _():
            pltpu.sync_copy(tok_hbm, tbuf)
            pltpu.sync_copy(tbuf, tok_out_hbm)
        ib = (ib0, ib1)
        mb = (mb0, mb1)
        ZR = CHUNK // 2
        ebase = b * E_PAD + s * per_tile
        nbase = s * out_rows

        def zrow(i, _):
            for t in range(D // 16):
                zero_v[i, pl.ds(t * 16, 16)] = jnp.zeros((16,), jnp.float32)
            return ()

        lax.fori_loop(0, ZR, zrow, ())

        zs = [pltpu.async_copy(
            zero_v, aggr_sp.at[pl.ds(nbase + j * ZR, ZR)], zsem)
            for j in range(2 * n_out)]
        for d in zs:
            d.wait()
        plsc.subcore_barrier()

        il = [None] * n_chunks
        ml = [None] * n_chunks
        il[0] = pltpu.async_copy(tgt_hbm.at[pl.ds(ebase, CHUNK)], ib0, lsem)
        ml[0] = pltpu.async_copy(msg_hbm.at[pl.ds(ebase, CHUNK)], mb0, lsem)
        for j in range(n_chunks):
            il[j].wait()
            ml[j].wait()
            if j + 1 < n_chunks:
                off = ebase + (j + 1) * CHUNK
                il[j + 1] = pltpu.async_copy(
                    tgt_hbm.at[pl.ds(off, CHUNK)], ib[(j + 1) % 2], lsem)
                ml[j + 1] = pltpu.async_copy(
                    msg_hbm.at[pl.ds(off, CHUNK)], mb[(j + 1) % 2], lsem)
            pltpu.sync_copy(mb[j % 2], aggr_sp.at[ib[j % 2]], add=True)
        plsc.subcore_barrier()

        wb = [None] * n_out
        for j in range(n_out):
            if j >= 2:
                wb[j - 2].wait()
            r0 = nbase + j * CHUNK
            pltpu.sync_copy(aggr_sp.at[pl.ds(r0, CHUNK)], mb[j % 2])
            wb[j] = pltpu.async_copy(
                mb[j % 2], out_hbm.at[b, pl.ds(r0, CHUNK)], wsem)
        wb[n_out - 2].wait()
        wb[n_out - 1].wait()

    return k(msg, tgt, tok)


def _msg0_body(hr_ref, cf_ref, sc_ref, cm_ref, em_ref, src_ref, rq_ref,
               bw_ref, bb_ref, w14_ref, w5_ref, v0_ref, b0_ref,
               msg_ref, gate_ref, hrbf_ref):
    hr = hr_ref[0]
    cf = cf_ref[0]
    hrbf_ref[0] = hr.astype(jnp.bfloat16)
    bwr = bw_ref[...]
    logit = (jnp.sum(hr * bwr, axis=1, keepdims=True)
             + jnp.sum(rq_ref[0] * bwr, axis=1, keepdims=True) + bb_ref[0, 0])
    beta = jax.nn.sigmoid(logit)
    gk = jax.nn.sigmoid((sc_ref[0] - beta) / TAU)
    gate = jnp.where(cm_ref[0] > 0, gk, 0.5) * em_ref[0]
    gate_ref[0] = gate
    is0 = (src_ref[0] == 0).astype(jnp.float32)
    t14 = jnp.dot(hr, w14_ref[...], preferred_element_type=jnp.float32)
    t5 = jnp.dot(cf, w5_ref[...], preferred_element_type=jnp.float32)
    z = is0 * (t14[:, :D] + v0_ref[...]) + t14[:, D:] + t5 + b0_ref[...]
    msg_ref[0] = gate * jnp.maximum(z, 0.0)


def _msgk_body(hs_ref, hr_ref, cf_ref, gate_ref, src_ref,
               w12_ref, w45_ref, w3s_ref, bk_ref, msg_ref):
    hs = hs_ref[0]
    hr = hr_ref[0]
    a = jnp.concatenate([hs * hr.astype(jnp.float32), hs], axis=1)
    bcat = jnp.concatenate([hr, cf_ref[0]], axis=1)
    z = (jnp.dot(a, w12_ref[...], preferred_element_type=jnp.float32)
         + jnp.dot(bcat, w45_ref[...], preferred_element_type=jnp.float32))
    is0 = (src_ref[0] == 0).astype(jnp.float32)
    z = z + is0 * w3s_ref[...] + bk_ref[...]
    msg_ref[0] = gate_ref[0] * jnp.maximum(z, 0.0)


def _msg2_body(hs_ref, hr_ref, cf_ref, gate_ref, src_ref, tgt_ref, h0_ref,
               w12_ref, w45_ref, w3s_ref, bk_ref, wu_ref, ub_ref,
               lg_ref, lb_ref, out_ref):
    g = pl.program_id(1)
    ng = pl.num_programs(1)
    hs = hs_ref[0]
    hr = hr_ref[0]
    a = jnp.concatenate([hs * hr.astype(jnp.float32), hs], axis=1)
    bcat = jnp.concatenate([hr, cf_ref[0]], axis=1)
    z = (jnp.dot(a, w12_ref[...], preferred_element_type=jnp.float32)
         + jnp.dot(bcat, w45_ref[...], preferred_element_type=jnp.float32))
    is0 = (src_ref[0] == 0).astype(jnp.float32)
    z = z + is0 * w3s_ref[...] + bk_ref[...]
    wmsg = gate_ref[0] * jnp.maximum(z, 0.0)
    t0 = (tgt_ref[0] == 0).astype(jnp.float32)
    part = jnp.sum(wmsg * t0, axis=0, keepdims=True)

    @pl.when(g == 0)
    def _():
        out_ref[0] = part

    @pl.when((g > 0) & (g < ng - 1))
    def _():
        out_ref[0] = out_ref[0] + part

    @pl.when(g == ng - 1)
    def _():
        aggr0 = out_ref[0] + part
        u = jnp.dot(aggr0, wu_ref[...],
                    preferred_element_type=jnp.float32) + ub_ref[...]
        x = h0_ref[0] + u
        m = jnp.mean(x, axis=1, keepdims=True)
        xc = x - m
        v = jnp.mean(xc * xc, axis=1, keepdims=True)
        out_ref[0] = xc * lax.rsqrt(v + 1e-5) * lg_ref[...] + lb_ref[...]


def _upd0_body(ag_ref, w_ref, ub_ref, lg_ref, lb_ref, out_ref):
    g = pl.program_id(1)
    u = jnp.dot(ag_ref[0], w_ref[...],
                preferred_element_type=jnp.float32) + ub_ref[...]
    rows = lax.broadcasted_iota(jnp.int32, (RN, D), 0)
    ind = ((rows == 0) & (g == 0)).astype(jnp.float32)
    x = ind + u
    m = jnp.mean(x, axis=1, keepdims=True)
    xc = x - m
    v = jnp.mean(xc * xc, axis=1, keepdims=True)
    out_ref[0] = xc * lax.rsqrt(v + 1e-5) * lg_ref[...] + lb_ref[...]


def _upd_body(ag_ref, h_ref, w_ref, ub_ref, lg_ref, lb_ref, out_ref):
    u = jnp.dot(ag_ref[0], w_ref[...],
                preferred_element_type=jnp.float32) + ub_ref[...]
    x = h_ref[0] + u
    m = jnp.mean(x, axis=1, keepdims=True)
    xc = x - m
    v = jnp.mean(xc * xc, axis=1, keepdims=True)
    out_ref[0] = xc * lax.rsqrt(v + 1e-5) * lg_ref[...] + lb_ref[...]


def kernel(edge_index, rels, scores, edge_conf_mask, edge_mask, mask,
           r_query_embed, conf_embeds, rel_embed_table, beta_w, beta_b,
           msg_w, msg_b, upd_w, upd_b, ln_g, ln_b):
    Bn, MaxN = mask.shape
    E = rels.shape[1]
    pe = E_PAD - E
    n_tiles = NC * NS

    src = jnp.pad(edge_index[:, 0, :], ((0, 0), (0, pe)))
    tgt = jnp.pad(edge_index[:, 1, :], ((0, 0), (0, pe)))
    rels_p = jnp.pad(rels, ((0, 0), (0, pe)))
    sc_c = jnp.pad(scores, ((0, 0), (0, pe)))[..., None]
    cm_c = jnp.pad(edge_conf_mask.astype(jnp.int32), ((0, 0), (0, pe)))[..., None]
    em_c = jnp.pad(edge_mask.astype(jnp.float32), ((0, 0), (0, pe)))[..., None]
    src_c = src[..., None]
    tgt_c = tgt[..., None]
    cf = jnp.pad(conf_embeds, ((0, 0), (0, pe), (0, 0))).astype(jnp.bfloat16)
    rq = r_query_embed[:, None, :]
    bw_row = beta_w.reshape(1, D)
    bb = beta_b.reshape(1, 1)

    rep = jnp.broadcast_to(
        rel_embed_table[None], (n_tiles,) + rel_embed_table.shape)
    rep = rep.reshape(-1, D)
    MA = Bn * E_PAD
    rpt = MA // n_tiles
    tile_off = (jnp.arange(MA, dtype=jnp.int32) // rpt) * N_RELS
    boff = jnp.arange(PB, dtype=jnp.int32)[:, None] * N_PAD

    w1 = msg_w[:, 0:D, :]
    w2s = jnp.sum(msg_w[:, D:2 * D, :], axis=1, keepdims=True)
    w3s = jnp.sum(msg_w[:, 2 * D:3 * D, :], axis=1, keepdims=True)
    w14_0 = jnp.concatenate([w1[0], msg_w[0, 3 * D:4 * D, :]], axis=1)
    w5_0 = msg_w[0, 4 * D:5 * D, :].astype(jnp.bfloat16)
    w12 = jnp.concatenate([w1, msg_w[:, D:2 * D, :]], axis=1)
    w45bf = msg_w[:, 3 * D:5 * D, :].astype(jnp.bfloat16)
    mb = msg_b[:, None, :]
    ub = upd_b[:, None, :]
    lg = ln_g.reshape(1, D)
    lb = ln_b.reshape(1, D)

    feat = pl.BlockSpec((1, R, D), lambda b, g: (b, g, 0))
    col = pl.BlockSpec((1, R, 1), lambda b, g: (b, g, 0))
    full2 = lambda shape: pl.BlockSpec(shape, lambda b, g: (0, 0))
    grid = (PB, E_PAD // R)
    node = pl.BlockSpec((1, RN, D), lambda b, g: (b, g, 0))
    node_grid = (PB, N_PAD // RN)
    rq_spec = pl.BlockSpec((1, 1, D), lambda b, g: (b, 0, 0))
    acc_spec = pl.BlockSpec((1, 1, D), lambda b, g: (b, 0, 0))

    tok = jnp.zeros((8,), jnp.int32)
    npair = Bn // PB
    rels_f = [None] * npair
    src_g = [None] * npair
    tgt_f = [None] * npair
    sls = [slice(q * PB, q * PB + PB) for q in range(npair)]
    for q in range(npair):
        src_g[q] = (src[sls[q]] + boff).reshape(-1)
        tgt_f[q] = tgt[sls[q]].reshape(-1)

    rels_all = rels_p.reshape(-1) + tile_off
    hr = [None] * npair
    for q in range(npair):
        half = Bn * E_PAD // npair
        hrq, tok = _sc_gather_rows(
            rep, rels_all[q * half:(q + 1) * half], tok)
        hr[q] = hrq.reshape(PB, E_PAD, D)

    msg0, gate, hr_bf = [None] * npair, [None] * npair, [None] * npair
    for q in range(npair):
        msg0[q], gate[q], hr_bf[q] = pl.pallas_call(
            _msg0_body,
            grid=grid,
            in_specs=[feat, feat, col, col, col, col, rq_spec,
                      full2((1, D)), full2((1, 1)), full2((D, 2 * D)),
                      full2((D, D)), full2((1, D)), full2((1, D))],
            out_specs=[feat, col, feat],
            out_shape=[jax.ShapeDtypeStruct((PB, E_PAD, D), jnp.float32),
                       jax.ShapeDtypeStruct((PB, E_PAD, 1), jnp.float32),
                       jax.ShapeDtypeStruct((PB, E_PAD, D), jnp.bfloat16)],
        )(hr[q], cf[sls[q]], sc_c[sls[q]], cm_c[sls[q]], em_c[sls[q]],
          src_c[sls[q]], rq[sls[q]], bw_row, bb, w14_0, w5_0,
          w2s[0] + w3s[0], mb[0])

    aggr0 = [None] * npair
    for q in range(npair):
        aggr0[q], tok = _sc_scatter_add(msg0[q].reshape(-1, D), tgt_f[q], tok)

    h1 = [None] * npair
    for q in range(npair):
        h1[q] = pl.pallas_call(
            _upd0_body,
            grid=node_grid,
            in_specs=[node, full2((D, D)), full2((1, D)),
                      full2((1, D)), full2((1, D))],
            out_specs=node,
            out_shape=jax.ShapeDtypeStruct((PB, N_PAD, D), jnp.float32),
        )(aggr0[q], upd_w[0], ub[0], lg, lb)

    hs1 = [None] * npair
    for q in range(npair):
        hs1[q], tok = _sc_gather_rows(h1[q].reshape(-1, D), src_g[q], tok)
        hs1[q] = hs1[q].reshape(PB, E_PAD, D)

    msg1 = [None] * npair
    for q in range(npair):
        msg1[q] = pl.pallas_call(
            _msgk_body,
            grid=grid,
            in_specs=[feat, feat, feat, col, col,
                      full2((2 * D, D)), full2((2 * D, D)), full2((1, D)),
                      full2((1, D))],
            out_specs=feat,
            out_shape=jax.ShapeDtypeStruct((PB, E_PAD, D), jnp.float32),
        )(hs1[q], hr_bf[q], cf[sls[q]], gate[q], src_c[sls[q]],
          w12[1], w45bf[1], w3s[1], mb[1])

    aggr1 = [None] * npair
    for q in range(npair):
        aggr1[q], tok = _sc_scatter_add(msg1[q].reshape(-1, D), tgt_f[q], tok)

    h2 = [None] * npair
    for q in range(npair):
        h2[q] = pl.pallas_call(
            _upd_body,
            grid=node_grid,
            in_specs=[node, node, full2((D, D)), full2((1, D)),
                      full2((1, D)), full2((1, D))],
            out_specs=node,
            out_shape=jax.ShapeDtypeStruct((PB, N_PAD, D), jnp.float32),
        )(aggr1[q], h1[q], upd_w[1], ub[1], lg, lb)

    hs2 = [None] * npair
    for q in range(npair):
        hs2[q], tok = _sc_gather_rows(h2[q].reshape(-1, D), src_g[q], tok)
        hs2[q] = hs2[q].reshape(PB, E_PAD, D)

    ctx = []
    for q in range(npair):
        ctx2 = pl.pallas_call(
            _msg2_body,
            grid=grid,
            in_specs=[feat, feat, feat, col, col, col, acc_spec,
                      full2((2 * D, D)), full2((2 * D, D)), full2((1, D)),
                      full2((1, D)), full2((D, D)), full2((1, D)),
                      full2((1, D)), full2((1, D))],
            out_specs=acc_spec,
            out_shape=jax.ShapeDtypeStruct((PB, 1, D), jnp.float32),
        )(hs2[q], hr_bf[q], cf[sls[q]], gate[q], src_c[sls[q]],
          tgt_c[sls[q]], h2[q][:, :1, :],
          w12[2], w45bf[2], w3s[2], mb[2], upd_w[2], ub[2], lg, lb)
        ctx.append(jnp.stack(
            [h1[q][:, 0, :], h2[q][:, 0, :], ctx2[:, 0, :]], axis=1))

    return jnp.concatenate(ctx, axis=0)


# speedup vs baseline: 5.0594x; 1.1620x over previous
import functools

import jax
import jax.numpy as jnp
from jax import lax
from jax.experimental import pallas as pl
from jax.experimental.pallas import tpu as pltpu
from jax.experimental.pallas import tpu_sc as plsc

D = 128
N_RELS = 512
TAU = 0.1
E_PAD = 20480
N_PAD = 10240
NC = 2
NS = 16
CHUNK = 128
R = 1024
RN = 1024
PB = 2


def _sc_gather_rows(table, idx, tok):
    M = idx.shape[0]
    dtype = table.dtype
    per_tile = M // (NC * NS)
    n_chunks = per_tile // CHUNK
    mesh = plsc.VectorSubcoreMesh(core_axis_name="c", subcore_axis_name="s")

    @functools.partial(
        pl.kernel,
        out_type=[jax.ShapeDtypeStruct((M, D), dtype),
                  jax.ShapeDtypeStruct((8,), jnp.int32)],
        mesh=mesh,
        scratch_types=[
            pltpu.VMEM((per_tile,), jnp.int32),
            pltpu.VMEM((CHUNK, D), dtype),
            pltpu.VMEM((CHUNK, D), dtype),
            pltpu.VMEM((CHUNK, D), dtype),
            pltpu.VMEM((CHUNK, D), dtype),
            pltpu.VMEM((8,), jnp.int32),
            pltpu.SemaphoreType.DMA,
            pltpu.SemaphoreType.DMA,
        ],
    )
    def k(table_hbm, idx_hbm, tok_hbm, out_hbm, tok_out_hbm,
          idx_all, rows0, rows1, rows2, rows3, tbuf, gsem, wsem):
        wid = lax.axis_index("s") * NC + lax.axis_index("c")
        base = wid * per_tile

        @pl.when(wid == 0)
        def _():
            pltpu.sync_copy(tok_hbm, tbuf)
            pltpu.sync_copy(tbuf, tok_out_hbm)
        pltpu.sync_copy(idx_hbm.at[pl.ds(base, per_tile)], idx_all)
        rows = (rows0, rows1, rows2, rows3)
        nd = len(rows)
        g = [None] * n_chunks
        w = [None] * n_chunks

        def fire(j):
            return pltpu.async_copy(
                table_hbm.at[idx_all.at[pl.ds(j * CHUNK, CHUNK)]],
                rows[j % nd], gsem)

        for j in range(min(nd - 1, n_chunks)):
            g[j] = fire(j)
        for j in range(n_chunks):
            g[j].wait()
            nxt = j + nd - 1
            if nxt < n_chunks:
                if j >= 1:
                    w[j - 1].wait()
                g[nxt] = fire(nxt)
            w[j] = pltpu.async_copy(
                rows[j % nd], out_hbm.at[pl.ds(base + j * CHUNK, CHUNK)], wsem)
        for j in range(max(0, n_chunks - (nd - 1)), n_chunks):
            w[j].wait()

    return k(table, idx, tok)


def _sc_scatter_add(msg, tgt, tok):
    per_tile = E_PAD // NS
    n_chunks = per_tile // CHUNK
    out_rows = N_PAD // NS
    n_out = out_rows // CHUNK
    ZR = CHUNK // 2
    mesh = plsc.VectorSubcoreMesh(core_axis_name="c", subcore_axis_name="s")

    @functools.partial(
        pl.kernel,
        out_type=[jax.ShapeDtypeStruct((PB, N_PAD, D), jnp.float32),
                  jax.ShapeDtypeStruct((8,), jnp.int32)],
        mesh=mesh,
        scratch_types=[
            pltpu.VMEM((CHUNK,), jnp.int32),
            pltpu.VMEM((CHUNK,), jnp.int32),
            pltpu.VMEM((CHUNK, D), jnp.float32),
            pltpu.VMEM((CHUNK, D), jnp.float32),
            pltpu.VMEM((CHUNK // 2, D), jnp.float32),
            pltpu.VMEM((8,), jnp.int32),
            pltpu.VMEM_SHARED((N_PAD, D), jnp.float32),
            pltpu.SemaphoreType.DMA,
            pltpu.SemaphoreType.DMA,
            pltpu.SemaphoreType.DMA,
        ],
    )
    def k(msg_hbm, tgt_hbm, tok_hbm, out_hbm, tok_out_hbm,
          ib0, ib1, mb0, mb1, zero_v, tbuf, aggr_sp, lsem, zsem, wsem):
        b = lax.axis_index("c")
        s = lax.axis_index("s")

        @pl.when((b == 0) & (s == 0))
        def _():
            pltpu.sync_copy(tok_hbm, tbuf)
            pltpu.sync_copy(tbuf, tok_out_hbm)
        ib = (ib0, ib1)
        mb = (mb0, mb1)
        ZR = CHUNK // 2
        ebase = b * E_PAD + s * per_tile
        nbase = s * out_rows

        def zrow(i, _):
            for t in range(D // 16):
                zero_v[i, pl.ds(t * 16, 16)] = jnp.zeros((16,), jnp.float32)
            return ()

        lax.fori_loop(0, ZR, zrow, ())

        zs = [pltpu.async_copy(
            zero_v, aggr_sp.at[pl.ds(nbase + j * ZR, ZR)], zsem)
            for j in range(2 * n_out)]
        for d in zs:
            d.wait()
        plsc.subcore_barrier()

        il = [None] * n_chunks
        ml = [None] * n_chunks
        il[0] = pltpu.async_copy(tgt_hbm.at[pl.ds(ebase, CHUNK)], ib0, lsem)
        ml[0] = pltpu.async_copy(msg_hbm.at[pl.ds(ebase, CHUNK)], mb0, lsem)
        for j in range(n_chunks):
            il[j].wait()
            ml[j].wait()
            if j + 1 < n_chunks:
                off = ebase + (j + 1) * CHUNK
                il[j + 1] = pltpu.async_copy(
                    tgt_hbm.at[pl.ds(off, CHUNK)], ib[(j + 1) % 2], lsem)
                ml[j + 1] = pltpu.async_copy(
                    msg_hbm.at[pl.ds(off, CHUNK)], mb[(j + 1) % 2], lsem)
            pltpu.sync_copy(mb[j % 2], aggr_sp.at[ib[j % 2]], add=True)
        plsc.subcore_barrier()

        wb = [None] * n_out
        for j in range(n_out):
            if j >= 2:
                wb[j - 2].wait()
            r0 = nbase + j * CHUNK
            pltpu.sync_copy(aggr_sp.at[pl.ds(r0, CHUNK)], mb[j % 2])
            wb[j] = pltpu.async_copy(
                mb[j % 2], out_hbm.at[b, pl.ds(r0, CHUNK)], wsem)
        wb[n_out - 2].wait()
        wb[n_out - 1].wait()

    return k(msg, tgt, tok)


def _msg0_body(hr_ref, cf_ref, sc_ref, cm_ref, em_ref, src_ref, rq_ref,
               bw_ref, bb_ref, w14_ref, w5_ref, v0_ref, b0_ref,
               msg_ref, gate_ref, hrbf_ref):
    hr = hr_ref[0]
    cf = cf_ref[0]
    hrbf_ref[0] = hr.astype(jnp.bfloat16)
    bwr = bw_ref[...]
    logit = (jnp.sum(hr * bwr, axis=1, keepdims=True)
             + jnp.sum(rq_ref[0] * bwr, axis=1, keepdims=True) + bb_ref[0, 0])
    beta = jax.nn.sigmoid(logit)
    gk = jax.nn.sigmoid((sc_ref[0] - beta) / TAU)
    gate = jnp.where(cm_ref[0] > 0, gk, 0.5) * em_ref[0]
    gate_ref[0] = gate
    is0 = (src_ref[0] == 0).astype(jnp.float32)
    t14 = jnp.dot(hr, w14_ref[...], preferred_element_type=jnp.float32)
    t5 = jnp.dot(cf, w5_ref[...], preferred_element_type=jnp.float32)
    z = is0 * (t14[:, :D] + v0_ref[...]) + t14[:, D:] + t5 + b0_ref[...]
    msg_ref[0] = gate * jnp.maximum(z, 0.0)


def _msgk_body(hs_ref, hr_ref, cf_ref, gate_ref, src_ref,
               w12_ref, w45_ref, w3s_ref, bk_ref, msg_ref):
    hs = hs_ref[0]
    hr = hr_ref[0]
    a = jnp.concatenate([hs * hr.astype(jnp.float32), hs], axis=1)
    bcat = jnp.concatenate([hr, cf_ref[0]], axis=1)
    z = (jnp.dot(a, w12_ref[...], preferred_element_type=jnp.float32)
         + jnp.dot(bcat, w45_ref[...], preferred_element_type=jnp.float32))
    is0 = (src_ref[0] == 0).astype(jnp.float32)
    z = z + is0 * w3s_ref[...] + bk_ref[...]
    msg_ref[0] = gate_ref[0] * jnp.maximum(z, 0.0)


def _msg2_body(hs_ref, hr_ref, cf_ref, gate_ref, src_ref, tgt_ref, h0_ref,
               w12_ref, w45_ref, w3s_ref, bk_ref, wu_ref, ub_ref,
               lg_ref, lb_ref, out_ref):
    g = pl.program_id(1)
    ng = pl.num_programs(1)
    hs = hs_ref[0]
    hr = hr_ref[0]
    a = jnp.concatenate([hs * hr.astype(jnp.float32), hs], axis=1)
    bcat = jnp.concatenate([hr, cf_ref[0]], axis=1)
    z = (jnp.dot(a, w12_ref[...], preferred_element_type=jnp.float32)
         + jnp.dot(bcat, w45_ref[...], preferred_element_type=jnp.float32))
    is0 = (src_ref[0] == 0).astype(jnp.float32)
    z = z + is0 * w3s_ref[...] + bk_ref[...]
    wmsg = gate_ref[0] * jnp.maximum(z, 0.0)
    t0 = (tgt_ref[0] == 0).astype(jnp.float32)
    part = jnp.sum(wmsg * t0, axis=0, keepdims=True)

    @pl.when(g == 0)
    def _():
        out_ref[0] = part

    @pl.when((g > 0) & (g < ng - 1))
    def _():
        out_ref[0] = out_ref[0] + part

    @pl.when(g == ng - 1)
    def _():
        aggr0 = out_ref[0] + part
        u = jnp.dot(aggr0, wu_ref[...],
                    preferred_element_type=jnp.float32) + ub_ref[...]
        x = h0_ref[0] + u
        m = jnp.mean(x, axis=1, keepdims=True)
        xc = x - m
        v = jnp.mean(xc * xc, axis=1, keepdims=True)
        out_ref[0] = xc * lax.rsqrt(v + 1e-5) * lg_ref[...] + lb_ref[...]


def _upd0_body(ag_ref, w_ref, ub_ref, lg_ref, lb_ref, out_ref):
    g = pl.program_id(1)
    u = jnp.dot(ag_ref[0], w_ref[...],
                preferred_element_type=jnp.float32) + ub_ref[...]
    rows = lax.broadcasted_iota(jnp.int32, (RN, D), 0)
    ind = ((rows == 0) & (g == 0)).astype(jnp.float32)
    x = ind + u
    m = jnp.mean(x, axis=1, keepdims=True)
    xc = x - m
    v = jnp.mean(xc * xc, axis=1, keepdims=True)
    out_ref[0] = xc * lax.rsqrt(v + 1e-5) * lg_ref[...] + lb_ref[...]


def _upd_body(ag_ref, h_ref, w_ref, ub_ref, lg_ref, lb_ref, out_ref):
    u = jnp.dot(ag_ref[0], w_ref[...],
                preferred_element_type=jnp.float32) + ub_ref[...]
    x = h_ref[0] + u
    m = jnp.mean(x, axis=1, keepdims=True)
    xc = x - m
    v = jnp.mean(xc * xc, axis=1, keepdims=True)
    out_ref[0] = xc * lax.rsqrt(v + 1e-5) * lg_ref[...] + lb_ref[...]


def kernel(edge_index, rels, scores, edge_conf_mask, edge_mask, mask,
           r_query_embed, conf_embeds, rel_embed_table, beta_w, beta_b,
           msg_w, msg_b, upd_w, upd_b, ln_g, ln_b):
    Bn, MaxN = mask.shape
    E = rels.shape[1]
    pe = E_PAD - E
    n_tiles = NC * NS

    src = jnp.pad(edge_index[:, 0, :], ((0, 0), (0, pe)))
    tgt = jnp.pad(edge_index[:, 1, :], ((0, 0), (0, pe)))
    rels_p = jnp.pad(rels, ((0, 0), (0, pe)))
    sc_c = jnp.pad(scores, ((0, 0), (0, pe)))[..., None]
    cm_c = jnp.pad(edge_conf_mask.astype(jnp.int32), ((0, 0), (0, pe)))[..., None]
    em_c = jnp.pad(edge_mask.astype(jnp.float32), ((0, 0), (0, pe)))[..., None]
    src_c = src[..., None]
    tgt_c = tgt[..., None]
    cf = jnp.pad(conf_embeds, ((0, 0), (0, pe), (0, 0))).astype(jnp.bfloat16)
    rq = r_query_embed[:, None, :]
    bw_row = beta_w.reshape(1, D)
    bb = beta_b.reshape(1, 1)

    rep = jnp.broadcast_to(
        rel_embed_table[None], (n_tiles,) + rel_embed_table.shape)
    rep = rep.reshape(-1, D)
    MA = Bn * E_PAD
    rpt = MA // n_tiles
    tile_off = (jnp.arange(MA, dtype=jnp.int32) // rpt) * N_RELS
    boff = jnp.arange(PB, dtype=jnp.int32)[:, None] * N_PAD

    w1 = msg_w[:, 0:D, :]
    w2s = jnp.sum(msg_w[:, D:2 * D, :], axis=1, keepdims=True)
    w3s = jnp.sum(msg_w[:, 2 * D:3 * D, :], axis=1, keepdims=True)
    w14_0 = jnp.concatenate([w1[0], msg_w[0, 3 * D:4 * D, :]], axis=1)
    w5_0 = msg_w[0, 4 * D:5 * D, :].astype(jnp.bfloat16)
    w12 = jnp.concatenate([w1, msg_w[:, D:2 * D, :]], axis=1)
    w45bf = msg_w[:, 3 * D:5 * D, :].astype(jnp.bfloat16)
    mb = msg_b[:, None, :]
    ub = upd_b[:, None, :]
    lg = ln_g.reshape(1, D)
    lb = ln_b.reshape(1, D)

    feat = pl.BlockSpec((1, R, D), lambda b, g: (b, g, 0))
    col = pl.BlockSpec((1, R, 1), lambda b, g: (b, g, 0))
    full2 = lambda shape: pl.BlockSpec(shape, lambda b, g: (0, 0))
    grid = (PB, E_PAD // R)
    node = pl.BlockSpec((1, RN, D), lambda b, g: (b, g, 0))
    node_grid = (PB, N_PAD // RN)
    rq_spec = pl.BlockSpec((1, 1, D), lambda b, g: (b, 0, 0))
    acc_spec = pl.BlockSpec((1, 1, D), lambda b, g: (b, 0, 0))

    tok = jnp.zeros((8,), jnp.int32)
    npair = Bn // PB
    rels_f = [None] * npair
    src_g = [None] * npair
    tgt_f = [None] * npair
    sls = [slice(q * PB, q * PB + PB) for q in range(npair)]
    for q in range(npair):
        src_g[q] = (src[sls[q]] + boff).reshape(-1)
        tgt_f[q] = tgt[sls[q]].reshape(-1)

    rels_all = rels_p.reshape(-1) + tile_off
    hr = [None] * npair
    for q in range(npair):
        half = Bn * E_PAD // npair
        hrq, tok = _sc_gather_rows(
            rep, rels_all[q * half:(q + 1) * half], tok)
        hr[q] = hrq.reshape(PB, E_PAD, D)

    msg0, gate, hr_bf = [None] * npair, [None] * npair, [None] * npair
    for q in range(npair):
        msg0[q], gate[q], hr_bf[q] = pl.pallas_call(
            _msg0_body,
            grid=grid,
            in_specs=[feat, feat, col, col, col, col, rq_spec,
                      full2((1, D)), full2((1, 1)), full2((D, 2 * D)),
                      full2((D, D)), full2((1, D)), full2((1, D))],
            out_specs=[feat, col, feat],
            out_shape=[jax.ShapeDtypeStruct((PB, E_PAD, D), jnp.float32),
                       jax.ShapeDtypeStruct((PB, E_PAD, 1), jnp.float32),
                       jax.ShapeDtypeStruct((PB, E_PAD, D), jnp.bfloat16)],
        )(hr[q], cf[sls[q]], sc_c[sls[q]], cm_c[sls[q]], em_c[sls[q]],
          src_c[sls[q]], rq[sls[q]], bw_row, bb, w14_0, w5_0,
          w2s[0] + w3s[0], mb[0])

    aggr0 = [None] * npair
    for q in range(npair):
        aggr0[q], tok = _sc_scatter_add(msg0[q].reshape(-1, D), tgt_f[q], tok)

    h1 = [None] * npair
    for q in range(npair):
        h1[q] = pl.pallas_call(
            _upd0_body,
            grid=node_grid,
            in_specs=[node, full2((D, D)), full2((1, D)),
                      full2((1, D)), full2((1, D))],
            out_specs=node,
            out_shape=jax.ShapeDtypeStruct((PB, N_PAD, D), jnp.float32),
        )(aggr0[q], upd_w[0], ub[0], lg, lb)

    hs1 = [None] * npair
    for q in range(npair):
        hs1[q], tok = _sc_gather_rows(h1[q].reshape(-1, D), src_g[q], tok)
        hs1[q] = hs1[q].reshape(PB, E_PAD, D)

    msg1 = [None] * npair
    for q in range(npair):
        msg1[q] = pl.pallas_call(
            _msgk_body,
            grid=grid,
            in_specs=[feat, feat, feat, col, col,
                      full2((2 * D, D)), full2((2 * D, D)), full2((1, D)),
                      full2((1, D))],
            out_specs=feat,
            out_shape=jax.ShapeDtypeStruct((PB, E_PAD, D), jnp.float32),
        )(hs1[q], hr_bf[q], cf[sls[q]], gate[q], src_c[sls[q]],
          w12[1], w45bf[1], w3s[1], mb[1])

    aggr1 = [None] * npair
    for q in range(npair):
        aggr1[q], tok = _sc_scatter_add(msg1[q].reshape(-1, D), tgt_f[q], tok)

    h2 = [None] * npair
    for q in range(npair):
        h2[q] = pl.pallas_call(
            _upd_body,
            grid=node_grid,
            in_specs=[node, node, full2((D, D)), full2((1, D)),
                      full2((1, D)), full2((1, D))],
            out_specs=node,
            out_shape=jax.ShapeDtypeStruct((PB, N_PAD, D), jnp.float32),
        )(aggr1[q], h1[q], upd_w[1], ub[1], lg, lb)

    hs2 = [None] * npair
    for q in range(npair):
        hs2[q], tok = _sc_gather_rows(h2[q].reshape(-1, D), src_g[q], tok)
        hs2[q] = hs2[q].reshape(PB, E_PAD, D)

    ctx = []
    for q in range(npair):
        ctx2 = pl.pallas_call(
            _msg2_body,
            grid=grid,
            in_specs=[feat, feat, feat, col, col, col, acc_spec,
                      full2((2 * D, D)), full2((2 * D, D)), full2((1, D)),
                      full2((1, D)), full2((D, D)), full2((1, D)),
                      full2((1, D)), full2((1, D))],
            out_specs=acc_spec,
            out_shape=jax.ShapeDtypeStruct((PB, 1, D), jnp.float32),
        )(hs2[q], hr_bf[q], cf[sls[q]], gate[q], src_c[sls[q]],
          tgt_c[sls[q]], h2[q][:, :1, :],
          w12[2], w45bf[2], w3s[2], mb[2], upd_w[2], ub[2], lg, lb)
        ctx.append(jnp.stack(
            [h1[q][:, 0, :], h2[q][:, 0, :], ctx2[:, 0, :]], axis=1))

    return jnp.concatenate(ctx, axis=0)


# speedup vs baseline: 5.4047x; 1.0683x over previous
import functools

import jax
import jax.numpy as jnp
from jax import lax
from jax.experimental import pallas as pl
from jax.experimental.pallas import tpu as pltpu
from jax.experimental.pallas import tpu_sc as plsc

D = 128
N_RELS = 512
TAU = 0.1
E_PAD = 20480
N_PAD = 10240
NC = 2
NS = 16
CHUNK = 128
R = 2048
RN = 2048
PB = 2


def _sc_gather_rows(table, idx, tok):
    M = idx.shape[0]
    dtype = table.dtype
    per_tile = M // (NC * NS)
    n_chunks = per_tile // CHUNK
    mesh = plsc.VectorSubcoreMesh(core_axis_name="c", subcore_axis_name="s")

    @functools.partial(
        pl.kernel,
        out_type=[jax.ShapeDtypeStruct((M, D), dtype),
                  jax.ShapeDtypeStruct((8,), jnp.int32)],
        mesh=mesh,
        scratch_types=[
            pltpu.VMEM((per_tile,), jnp.int32),
            pltpu.VMEM((CHUNK, D), dtype),
            pltpu.VMEM((CHUNK, D), dtype),
            pltpu.VMEM((CHUNK, D), dtype),
            pltpu.VMEM((CHUNK, D), dtype),
            pltpu.VMEM((8,), jnp.int32),
            pltpu.SemaphoreType.DMA,
            pltpu.SemaphoreType.DMA,
        ],
    )
    def k(table_hbm, idx_hbm, tok_hbm, out_hbm, tok_out_hbm,
          idx_all, rows0, rows1, rows2, rows3, tbuf, gsem, wsem):
        wid = lax.axis_index("s") * NC + lax.axis_index("c")
        base = wid * per_tile

        @pl.when(wid == 0)
        def _():
            pltpu.sync_copy(tok_hbm, tbuf)
            pltpu.sync_copy(tbuf, tok_out_hbm)
        pltpu.sync_copy(idx_hbm.at[pl.ds(base, per_tile)], idx_all)
        rows = (rows0, rows1, rows2, rows3)
        nd = len(rows)
        g = [None] * n_chunks
        w = [None] * n_chunks

        def fire(j):
            return pltpu.async_copy(
                table_hbm.at[idx_all.at[pl.ds(j * CHUNK, CHUNK)]],
                rows[j % nd], gsem)

        for j in range(min(nd - 1, n_chunks)):
            g[j] = fire(j)
        for j in range(n_chunks):
            g[j].wait()
            nxt = j + nd - 1
            if nxt < n_chunks:
                if j >= 1:
                    w[j - 1].wait()
                g[nxt] = fire(nxt)
            w[j] = pltpu.async_copy(
                rows[j % nd], out_hbm.at[pl.ds(base + j * CHUNK, CHUNK)], wsem)
        for j in range(max(0, n_chunks - (nd - 1)), n_chunks):
            w[j].wait()

    return k(table, idx, tok)


def _sc_scatter_add(msg, tgt, tok):
    per_tile = E_PAD // NS
    n_chunks = per_tile // CHUNK
    out_rows = N_PAD // NS
    n_out = out_rows // CHUNK
    ZR = CHUNK // 2
    mesh = plsc.VectorSubcoreMesh(core_axis_name="c", subcore_axis_name="s")

    @functools.partial(
        pl.kernel,
        out_type=[jax.ShapeDtypeStruct((PB, N_PAD, D), jnp.float32),
                  jax.ShapeDtypeStruct((8,), jnp.int32)],
        mesh=mesh,
        scratch_types=[
            pltpu.VMEM((CHUNK,), jnp.int32),
            pltpu.VMEM((CHUNK,), jnp.int32),
            pltpu.VMEM((CHUNK, D), jnp.float32),
            pltpu.VMEM((CHUNK, D), jnp.float32),
            pltpu.VMEM((CHUNK // 2, D), jnp.float32),
            pltpu.VMEM((8,), jnp.int32),
            pltpu.VMEM_SHARED((N_PAD, D), jnp.float32),
            pltpu.SemaphoreType.DMA,
            pltpu.SemaphoreType.DMA,
            pltpu.SemaphoreType.DMA,
        ],
    )
    def k(msg_hbm, tgt_hbm, tok_hbm, out_hbm, tok_out_hbm,
          ib0, ib1, mb0, mb1, zero_v, tbuf, aggr_sp, lsem, zsem, wsem):
        b = lax.axis_index("c")
        s = lax.axis_index("s")

        @pl.when((b == 0) & (s == 0))
        def _():
            pltpu.sync_copy(tok_hbm, tbuf)
            pltpu.sync_copy(tbuf, tok_out_hbm)
        ib = (ib0, ib1)
        mb = (mb0, mb1)
        ZR = CHUNK // 2
        ebase = b * E_PAD + s * per_tile
        nbase = s * out_rows

        def zrow(i, _):
            for t in range(D // 16):
                zero_v[i, pl.ds(t * 16, 16)] = jnp.zeros((16,), jnp.float32)
            return ()

        lax.fori_loop(0, ZR, zrow, ())

        zs = [pltpu.async_copy(
            zero_v, aggr_sp.at[pl.ds(nbase + j * ZR, ZR)], zsem)
            for j in range(2 * n_out)]
        for d in zs:
            d.wait()
        plsc.subcore_barrier()

        il = [None] * n_chunks
        ml = [None] * n_chunks
        il[0] = pltpu.async_copy(tgt_hbm.at[pl.ds(ebase, CHUNK)], ib0, lsem)
        ml[0] = pltpu.async_copy(msg_hbm.at[pl.ds(ebase, CHUNK)], mb0, lsem)
        for j in range(n_chunks):
            il[j].wait()
            ml[j].wait()
            if j + 1 < n_chunks:
                off = ebase + (j + 1) * CHUNK
                il[j + 1] = pltpu.async_copy(
                    tgt_hbm.at[pl.ds(off, CHUNK)], ib[(j + 1) % 2], lsem)
                ml[j + 1] = pltpu.async_copy(
                    msg_hbm.at[pl.ds(off, CHUNK)], mb[(j + 1) % 2], lsem)
            pltpu.sync_copy(mb[j % 2], aggr_sp.at[ib[j % 2]], add=True)
        plsc.subcore_barrier()

        wb = [None] * n_out
        for j in range(n_out):
            if j >= 2:
                wb[j - 2].wait()
            r0 = nbase + j * CHUNK
            pltpu.sync_copy(aggr_sp.at[pl.ds(r0, CHUNK)], mb[j % 2])
            wb[j] = pltpu.async_copy(
                mb[j % 2], out_hbm.at[b, pl.ds(r0, CHUNK)], wsem)
        wb[n_out - 2].wait()
        wb[n_out - 1].wait()

    return k(msg, tgt, tok)


def _msg0_body(hr_ref, cf_ref, sc_ref, cm_ref, em_ref, src_ref, rq_ref,
               bw_ref, bb_ref, w14_ref, w5_ref, v0_ref, b0_ref,
               msg_ref, gate_ref, hrbf_ref):
    hr = hr_ref[0]
    cf = cf_ref[0]
    hrbf_ref[0] = hr.astype(jnp.bfloat16)
    bwr = bw_ref[...]
    logit = (jnp.sum(hr * bwr, axis=1, keepdims=True)
             + jnp.sum(rq_ref[0] * bwr, axis=1, keepdims=True) + bb_ref[0, 0])
    beta = jax.nn.sigmoid(logit)
    gk = jax.nn.sigmoid((sc_ref[0] - beta) / TAU)
    gate = jnp.where(cm_ref[0] > 0, gk, 0.5) * em_ref[0]
    gate_ref[0] = gate
    is0 = (src_ref[0] == 0).astype(jnp.float32)
    t14 = jnp.dot(hr, w14_ref[...], preferred_element_type=jnp.float32)
    t5 = jnp.dot(cf, w5_ref[...], preferred_element_type=jnp.float32)
    z = is0 * (t14[:, :D] + v0_ref[...]) + t14[:, D:] + t5 + b0_ref[...]
    msg_ref[0] = gate * jnp.maximum(z, 0.0)


def _msgk_body(hs_ref, hr_ref, cf_ref, gate_ref, src_ref,
               w12_ref, w45_ref, w3s_ref, bk_ref, msg_ref):
    hs = hs_ref[0]
    hr = hr_ref[0]
    a = jnp.concatenate([hs * hr.astype(jnp.float32), hs], axis=1)
    bcat = jnp.concatenate([hr, cf_ref[0]], axis=1)
    z = (jnp.dot(a, w12_ref[...], preferred_element_type=jnp.float32)
         + jnp.dot(bcat, w45_ref[...], preferred_element_type=jnp.float32))
    is0 = (src_ref[0] == 0).astype(jnp.float32)
    z = z + is0 * w3s_ref[...] + bk_ref[...]
    msg_ref[0] = gate_ref[0] * jnp.maximum(z, 0.0)


def _msg2_body(hs_ref, hr_ref, cf_ref, gate_ref, src_ref, tgt_ref, h0_ref,
               w12_ref, w45_ref, w3s_ref, bk_ref, wu_ref, ub_ref,
               lg_ref, lb_ref, out_ref):
    g = pl.program_id(1)
    ng = pl.num_programs(1)
    hs = hs_ref[0]
    hr = hr_ref[0]
    a = jnp.concatenate([hs * hr.astype(jnp.float32), hs], axis=1)
    bcat = jnp.concatenate([hr, cf_ref[0]], axis=1)
    z = (jnp.dot(a, w12_ref[...], preferred_element_type=jnp.float32)
         + jnp.dot(bcat, w45_ref[...], preferred_element_type=jnp.float32))
    is0 = (src_ref[0] == 0).astype(jnp.float32)
    z = z + is0 * w3s_ref[...] + bk_ref[...]
    wmsg = gate_ref[0] * jnp.maximum(z, 0.0)
    t0 = (tgt_ref[0] == 0).astype(jnp.float32)
    part = jnp.sum(wmsg * t0, axis=0, keepdims=True)

    @pl.when(g == 0)
    def _():
        out_ref[0] = part

    @pl.when((g > 0) & (g < ng - 1))
    def _():
        out_ref[0] = out_ref[0] + part

    @pl.when(g == ng - 1)
    def _():
        aggr0 = out_ref[0] + part
        u = jnp.dot(aggr0, wu_ref[...],
                    preferred_element_type=jnp.float32) + ub_ref[...]
        x = h0_ref[0] + u
        m = jnp.mean(x, axis=1, keepdims=True)
        xc = x - m
        v = jnp.mean(xc * xc, axis=1, keepdims=True)
        out_ref[0] = xc * lax.rsqrt(v + 1e-5) * lg_ref[...] + lb_ref[...]


def _upd0_body(ag_ref, w_ref, ub_ref, lg_ref, lb_ref, out_ref):
    g = pl.program_id(1)
    u = jnp.dot(ag_ref[0], w_ref[...],
                preferred_element_type=jnp.float32) + ub_ref[...]
    rows = lax.broadcasted_iota(jnp.int32, (RN, D), 0)
    ind = ((rows == 0) & (g == 0)).astype(jnp.float32)
    x = ind + u
    m = jnp.mean(x, axis=1, keepdims=True)
    xc = x - m
    v = jnp.mean(xc * xc, axis=1, keepdims=True)
    out_ref[0] = xc * lax.rsqrt(v + 1e-5) * lg_ref[...] + lb_ref[...]


def _upd_body(ag_ref, h_ref, w_ref, ub_ref, lg_ref, lb_ref, out_ref):
    u = jnp.dot(ag_ref[0], w_ref[...],
                preferred_element_type=jnp.float32) + ub_ref[...]
    x = h_ref[0] + u
    m = jnp.mean(x, axis=1, keepdims=True)
    xc = x - m
    v = jnp.mean(xc * xc, axis=1, keepdims=True)
    out_ref[0] = xc * lax.rsqrt(v + 1e-5) * lg_ref[...] + lb_ref[...]


def kernel(edge_index, rels, scores, edge_conf_mask, edge_mask, mask,
           r_query_embed, conf_embeds, rel_embed_table, beta_w, beta_b,
           msg_w, msg_b, upd_w, upd_b, ln_g, ln_b):
    Bn, MaxN = mask.shape
    E = rels.shape[1]
    pe = E_PAD - E
    n_tiles = NC * NS

    src = jnp.pad(edge_index[:, 0, :], ((0, 0), (0, pe)))
    tgt = jnp.pad(edge_index[:, 1, :], ((0, 0), (0, pe)))
    rels_p = jnp.pad(rels, ((0, 0), (0, pe)))
    sc_c = jnp.pad(scores, ((0, 0), (0, pe)))[..., None]
    cm_c = jnp.pad(edge_conf_mask.astype(jnp.int32), ((0, 0), (0, pe)))[..., None]
    em_c = jnp.pad(edge_mask.astype(jnp.float32), ((0, 0), (0, pe)))[..., None]
    src_c = src[..., None]
    tgt_c = tgt[..., None]
    cf = jnp.pad(conf_embeds, ((0, 0), (0, pe), (0, 0))).astype(jnp.bfloat16)
    rq = r_query_embed[:, None, :]
    bw_row = beta_w.reshape(1, D)
    bb = beta_b.reshape(1, 1)

    rep = jnp.broadcast_to(
        rel_embed_table[None], (n_tiles,) + rel_embed_table.shape)
    rep = rep.reshape(-1, D)
    MA = Bn * E_PAD
    rpt = MA // n_tiles
    tile_off = (jnp.arange(MA, dtype=jnp.int32) // rpt) * N_RELS
    boff = jnp.arange(PB, dtype=jnp.int32)[:, None] * N_PAD

    w1 = msg_w[:, 0:D, :]
    w2s = jnp.sum(msg_w[:, D:2 * D, :], axis=1, keepdims=True)
    w3s = jnp.sum(msg_w[:, 2 * D:3 * D, :], axis=1, keepdims=True)
    w14_0 = jnp.concatenate([w1[0], msg_w[0, 3 * D:4 * D, :]], axis=1)
    w5_0 = msg_w[0, 4 * D:5 * D, :].astype(jnp.bfloat16)
    w12 = jnp.concatenate([w1, msg_w[:, D:2 * D, :]], axis=1)
    w45bf = msg_w[:, 3 * D:5 * D, :].astype(jnp.bfloat16)
    mb = msg_b[:, None, :]
    ub = upd_b[:, None, :]
    lg = ln_g.reshape(1, D)
    lb = ln_b.reshape(1, D)

    feat = pl.BlockSpec((1, R, D), lambda b, g: (b, g, 0))
    col = pl.BlockSpec((1, R, 1), lambda b, g: (b, g, 0))
    full2 = lambda shape: pl.BlockSpec(shape, lambda b, g: (0, 0))
    grid = (PB, E_PAD // R)
    node = pl.BlockSpec((1, RN, D), lambda b, g: (b, g, 0))
    node_grid = (PB, N_PAD // RN)
    rq_spec = pl.BlockSpec((1, 1, D), lambda b, g: (b, 0, 0))
    acc_spec = pl.BlockSpec((1, 1, D), lambda b, g: (b, 0, 0))

    tok = jnp.zeros((8,), jnp.int32)
    npair = Bn // PB
    rels_f = [None] * npair
    src_g = [None] * npair
    tgt_f = [None] * npair
    sls = [slice(q * PB, q * PB + PB) for q in range(npair)]
    for q in range(npair):
        src_g[q] = (src[sls[q]] + boff).reshape(-1)
        tgt_f[q] = tgt[sls[q]].reshape(-1)

    rels_all = rels_p.reshape(-1) + tile_off
    hr = [None] * npair
    for q in range(npair):
        half = Bn * E_PAD // npair
        hrq, tok = _sc_gather_rows(
            rep, rels_all[q * half:(q + 1) * half], tok)
        hr[q] = hrq.reshape(PB, E_PAD, D)

    msg0, gate, hr_bf = [None] * npair, [None] * npair, [None] * npair
    for q in range(npair):
        msg0[q], gate[q], hr_bf[q] = pl.pallas_call(
            _msg0_body,
            grid=grid,
            in_specs=[feat, feat, col, col, col, col, rq_spec,
                      full2((1, D)), full2((1, 1)), full2((D, 2 * D)),
                      full2((D, D)), full2((1, D)), full2((1, D))],
            out_specs=[feat, col, feat],
            out_shape=[jax.ShapeDtypeStruct((PB, E_PAD, D), jnp.float32),
                       jax.ShapeDtypeStruct((PB, E_PAD, 1), jnp.float32),
                       jax.ShapeDtypeStruct((PB, E_PAD, D), jnp.bfloat16)],
        )(hr[q], cf[sls[q]], sc_c[sls[q]], cm_c[sls[q]], em_c[sls[q]],
          src_c[sls[q]], rq[sls[q]], bw_row, bb, w14_0, w5_0,
          w2s[0] + w3s[0], mb[0])

    aggr0 = [None] * npair
    for q in range(npair):
        aggr0[q], tok = _sc_scatter_add(msg0[q].reshape(-1, D), tgt_f[q], tok)

    h1 = [None] * npair
    for q in range(npair):
        h1[q] = pl.pallas_call(
            _upd0_body,
            grid=node_grid,
            in_specs=[node, full2((D, D)), full2((1, D)),
                      full2((1, D)), full2((1, D))],
            out_specs=node,
            out_shape=jax.ShapeDtypeStruct((PB, N_PAD, D), jnp.float32),
        )(aggr0[q], upd_w[0], ub[0], lg, lb)

    hs1 = [None] * npair
    for q in range(npair):
        hs1[q], tok = _sc_gather_rows(h1[q].reshape(-1, D), src_g[q], tok)
        hs1[q] = hs1[q].reshape(PB, E_PAD, D)

    msg1 = [None] * npair
    for q in range(npair):
        msg1[q] = pl.pallas_call(
            _msgk_body,
            grid=grid,
            in_specs=[feat, feat, feat, col, col,
                      full2((2 * D, D)), full2((2 * D, D)), full2((1, D)),
                      full2((1, D))],
            out_specs=feat,
            out_shape=jax.ShapeDtypeStruct((PB, E_PAD, D), jnp.float32),
        )(hs1[q], hr_bf[q], cf[sls[q]], gate[q], src_c[sls[q]],
          w12[1], w45bf[1], w3s[1], mb[1])

    aggr1 = [None] * npair
    for q in range(npair):
        aggr1[q], tok = _sc_scatter_add(msg1[q].reshape(-1, D), tgt_f[q], tok)

    h2 = [None] * npair
    for q in range(npair):
        h2[q] = pl.pallas_call(
            _upd_body,
            grid=node_grid,
            in_specs=[node, node, full2((D, D)), full2((1, D)),
                      full2((1, D)), full2((1, D))],
            out_specs=node,
            out_shape=jax.ShapeDtypeStruct((PB, N_PAD, D), jnp.float32),
        )(aggr1[q], h1[q], upd_w[1], ub[1], lg, lb)

    hs2 = [None] * npair
    for q in range(npair):
        hs2[q], tok = _sc_gather_rows(h2[q].reshape(-1, D), src_g[q], tok)
        hs2[q] = hs2[q].reshape(PB, E_PAD, D)

    ctx = []
    for q in range(npair):
        ctx2 = pl.pallas_call(
            _msg2_body,
            grid=grid,
            in_specs=[feat, feat, feat, col, col, col, acc_spec,
                      full2((2 * D, D)), full2((2 * D, D)), full2((1, D)),
                      full2((1, D)), full2((D, D)), full2((1, D)),
                      full2((1, D)), full2((1, D))],
            out_specs=acc_spec,
            out_shape=jax.ShapeDtypeStruct((PB, 1, D), jnp.float32),
        )(hs2[q], hr_bf[q], cf[sls[q]], gate[q], src_c[sls[q]],
          tgt_c[sls[q]], h2[q][:, :1, :],
          w12[2], w45bf[2], w3s[2], mb[2], upd_w[2], ub[2], lg, lb)
        ctx.append(jnp.stack(
            [h1[q][:, 0, :], h2[q][:, 0, :], ctx2[:, 0, :]], axis=1))

    return jnp.concatenate(ctx, axis=0)


# speedup vs baseline: 5.4986x; 1.0174x over previous
import functools

import jax
import jax.numpy as jnp
from jax import lax
from jax.experimental import pallas as pl
from jax.experimental.pallas import tpu as pltpu
from jax.experimental.pallas import tpu_sc as plsc

D = 128
N_RELS = 512
TAU = 0.1
E_PAD = 20480
N_PAD = 10240
NC = 2
NS = 16
CHUNK = 128
R = 4096
RN = 2048
PB = 2


def _sc_gather_rows(table, idx, tok):
    M = idx.shape[0]
    dtype = table.dtype
    per_tile = M // (NC * NS)
    n_chunks = per_tile // CHUNK
    mesh = plsc.VectorSubcoreMesh(core_axis_name="c", subcore_axis_name="s")

    @functools.partial(
        pl.kernel,
        out_type=[jax.ShapeDtypeStruct((M, D), dtype),
                  jax.ShapeDtypeStruct((8,), jnp.int32)],
        mesh=mesh,
        scratch_types=[
            pltpu.VMEM((per_tile,), jnp.int32),
            pltpu.VMEM((CHUNK, D), dtype),
            pltpu.VMEM((CHUNK, D), dtype),
            pltpu.VMEM((CHUNK, D), dtype),
            pltpu.VMEM((CHUNK, D), dtype),
            pltpu.VMEM((8,), jnp.int32),
            pltpu.SemaphoreType.DMA,
            pltpu.SemaphoreType.DMA,
        ],
    )
    def k(table_hbm, idx_hbm, tok_hbm, out_hbm, tok_out_hbm,
          idx_all, rows0, rows1, rows2, rows3, tbuf, gsem, wsem):
        wid = lax.axis_index("s") * NC + lax.axis_index("c")
        base = wid * per_tile

        @pl.when(wid == 0)
        def _():
            pltpu.sync_copy(tok_hbm, tbuf)
            pltpu.sync_copy(tbuf, tok_out_hbm)
        pltpu.sync_copy(idx_hbm.at[pl.ds(base, per_tile)], idx_all)
        rows = (rows0, rows1, rows2, rows3)
        nd = len(rows)
        g = [None] * n_chunks
        w = [None] * n_chunks

        def fire(j):
            return pltpu.async_copy(
                table_hbm.at[idx_all.at[pl.ds(j * CHUNK, CHUNK)]],
                rows[j % nd], gsem)

        for j in range(min(nd - 1, n_chunks)):
            g[j] = fire(j)
        for j in range(n_chunks):
            g[j].wait()
            nxt = j + nd - 1
            if nxt < n_chunks:
                if j >= 1:
                    w[j - 1].wait()
                g[nxt] = fire(nxt)
            w[j] = pltpu.async_copy(
                rows[j % nd], out_hbm.at[pl.ds(base + j * CHUNK, CHUNK)], wsem)
        for j in range(max(0, n_chunks - (nd - 1)), n_chunks):
            w[j].wait()

    return k(table, idx, tok)


def _sc_scatter_add(msg, tgt, tok):
    per_tile = E_PAD // NS
    n_chunks = per_tile // CHUNK
    out_rows = N_PAD // NS
    n_out = out_rows // CHUNK
    ZR = CHUNK // 2
    mesh = plsc.VectorSubcoreMesh(core_axis_name="c", subcore_axis_name="s")

    @functools.partial(
        pl.kernel,
        out_type=[jax.ShapeDtypeStruct((PB, N_PAD, D), jnp.float32),
                  jax.ShapeDtypeStruct((8,), jnp.int32)],
        mesh=mesh,
        scratch_types=[
            pltpu.VMEM((CHUNK,), jnp.int32),
            pltpu.VMEM((CHUNK,), jnp.int32),
            pltpu.VMEM((CHUNK, D), jnp.float32),
            pltpu.VMEM((CHUNK, D), jnp.float32),
            pltpu.VMEM((CHUNK // 2, D), jnp.float32),
            pltpu.VMEM((8,), jnp.int32),
            pltpu.VMEM_SHARED((N_PAD, D), jnp.float32),
            pltpu.SemaphoreType.DMA,
            pltpu.SemaphoreType.DMA,
            pltpu.SemaphoreType.DMA,
        ],
    )
    def k(msg_hbm, tgt_hbm, tok_hbm, out_hbm, tok_out_hbm,
          ib0, ib1, mb0, mb1, zero_v, tbuf, aggr_sp, lsem, zsem, wsem):
        b = lax.axis_index("c")
        s = lax.axis_index("s")

        @pl.when((b == 0) & (s == 0))
        def _():
            pltpu.sync_copy(tok_hbm, tbuf)
            pltpu.sync_copy(tbuf, tok_out_hbm)
        ib = (ib0, ib1)
        mb = (mb0, mb1)
        ZR = CHUNK // 2
        ebase = b * E_PAD + s * per_tile
        nbase = s * out_rows

        def zrow(i, _):
            for t in range(D // 16):
                zero_v[i, pl.ds(t * 16, 16)] = jnp.zeros((16,), jnp.float32)
            return ()

        lax.fori_loop(0, ZR, zrow, ())

        zs = [pltpu.async_copy(
            zero_v, aggr_sp.at[pl.ds(nbase + j * ZR, ZR)], zsem)
            for j in range(2 * n_out)]
        for d in zs:
            d.wait()
        plsc.subcore_barrier()

        il = [None] * n_chunks
        ml = [None] * n_chunks
        il[0] = pltpu.async_copy(tgt_hbm.at[pl.ds(ebase, CHUNK)], ib0, lsem)
        ml[0] = pltpu.async_copy(msg_hbm.at[pl.ds(ebase, CHUNK)], mb0, lsem)
        for j in range(n_chunks):
            il[j].wait()
            ml[j].wait()
            if j + 1 < n_chunks:
                off = ebase + (j + 1) * CHUNK
                il[j + 1] = pltpu.async_copy(
                    tgt_hbm.at[pl.ds(off, CHUNK)], ib[(j + 1) % 2], lsem)
                ml[j + 1] = pltpu.async_copy(
                    msg_hbm.at[pl.ds(off, CHUNK)], mb[(j + 1) % 2], lsem)
            pltpu.sync_copy(mb[j % 2], aggr_sp.at[ib[j % 2]], add=True)
        plsc.subcore_barrier()

        wb = [None] * n_out
        for j in range(n_out):
            if j >= 2:
                wb[j - 2].wait()
            r0 = nbase + j * CHUNK
            pltpu.sync_copy(aggr_sp.at[pl.ds(r0, CHUNK)], mb[j % 2])
            wb[j] = pltpu.async_copy(
                mb[j % 2], out_hbm.at[b, pl.ds(r0, CHUNK)], wsem)
        wb[n_out - 2].wait()
        wb[n_out - 1].wait()

    return k(msg, tgt, tok)


def _msg0_body(hr_ref, cf_ref, sc_ref, cm_ref, em_ref, src_ref, rq_ref,
               bw_ref, bb_ref, w14_ref, w5_ref, v0_ref, b0_ref,
               msg_ref, gate_ref, hrbf_ref):
    hr = hr_ref[0]
    cf = cf_ref[0]
    hrbf_ref[0] = hr.astype(jnp.bfloat16)
    bwr = bw_ref[...]
    logit = (jnp.sum(hr * bwr, axis=1, keepdims=True)
             + jnp.sum(rq_ref[0] * bwr, axis=1, keepdims=True) + bb_ref[0, 0])
    beta = jax.nn.sigmoid(logit)
    gk = jax.nn.sigmoid((sc_ref[0] - beta) / TAU)
    gate = jnp.where(cm_ref[0] > 0, gk, 0.5) * em_ref[0]
    gate_ref[0] = gate
    is0 = (src_ref[0] == 0).astype(jnp.float32)
    t14 = jnp.dot(hr, w14_ref[...], preferred_element_type=jnp.float32)
    t5 = jnp.dot(cf, w5_ref[...], preferred_element_type=jnp.float32)
    z = is0 * (t14[:, :D] + v0_ref[...]) + t14[:, D:] + t5 + b0_ref[...]
    msg_ref[0] = gate * jnp.maximum(z, 0.0)


def _msgk_body(hs_ref, hr_ref, cf_ref, gate_ref, src_ref,
               w12_ref, w45_ref, w3s_ref, bk_ref, msg_ref):
    hs = hs_ref[0]
    hr = hr_ref[0]
    a = jnp.concatenate([hs * hr.astype(jnp.float32), hs], axis=1)
    bcat = jnp.concatenate([hr, cf_ref[0]], axis=1)
    z = (jnp.dot(a, w12_ref[...], preferred_element_type=jnp.float32)
         + jnp.dot(bcat, w45_ref[...], preferred_element_type=jnp.float32))
    is0 = (src_ref[0] == 0).astype(jnp.float32)
    z = z + is0 * w3s_ref[...] + bk_ref[...]
    msg_ref[0] = gate_ref[0] * jnp.maximum(z, 0.0)


def _msg2_body(hs_ref, hr_ref, cf_ref, gate_ref, src_ref, tgt_ref, h0_ref,
               w12_ref, w45_ref, w3s_ref, bk_ref, wu_ref, ub_ref,
               lg_ref, lb_ref, out_ref):
    g = pl.program_id(1)
    ng = pl.num_programs(1)
    hs = hs_ref[0]
    hr = hr_ref[0]
    a = jnp.concatenate([hs * hr.astype(jnp.float32), hs], axis=1)
    bcat = jnp.concatenate([hr, cf_ref[0]], axis=1)
    z = (jnp.dot(a, w12_ref[...], preferred_element_type=jnp.float32)
         + jnp.dot(bcat, w45_ref[...], preferred_element_type=jnp.float32))
    is0 = (src_ref[0] == 0).astype(jnp.float32)
    z = z + is0 * w3s_ref[...] + bk_ref[...]
    wmsg = gate_ref[0] * jnp.maximum(z, 0.0)
    t0 = (tgt_ref[0] == 0).astype(jnp.float32)
    part = jnp.sum(wmsg * t0, axis=0, keepdims=True)

    @pl.when(g == 0)
    def _():
        out_ref[0] = part

    @pl.when((g > 0) & (g < ng - 1))
    def _():
        out_ref[0] = out_ref[0] + part

    @pl.when(g == ng - 1)
    def _():
        aggr0 = out_ref[0] + part
        u = jnp.dot(aggr0, wu_ref[...],
                    preferred_element_type=jnp.float32) + ub_ref[...]
        x = h0_ref[0] + u
        m = jnp.mean(x, axis=1, keepdims=True)
        xc = x - m
        v = jnp.mean(xc * xc, axis=1, keepdims=True)
        out_ref[0] = xc * lax.rsqrt(v + 1e-5) * lg_ref[...] + lb_ref[...]


def _upd0_body(ag_ref, w_ref, ub_ref, lg_ref, lb_ref, out_ref):
    g = pl.program_id(1)
    u = jnp.dot(ag_ref[0], w_ref[...],
                preferred_element_type=jnp.float32) + ub_ref[...]
    rows = lax.broadcasted_iota(jnp.int32, (RN, D), 0)
    ind = ((rows == 0) & (g == 0)).astype(jnp.float32)
    x = ind + u
    m = jnp.mean(x, axis=1, keepdims=True)
    xc = x - m
    v = jnp.mean(xc * xc, axis=1, keepdims=True)
    out_ref[0] = xc * lax.rsqrt(v + 1e-5) * lg_ref[...] + lb_ref[...]


def _upd_body(ag_ref, h_ref, w_ref, ub_ref, lg_ref, lb_ref, out_ref):
    u = jnp.dot(ag_ref[0], w_ref[...],
                preferred_element_type=jnp.float32) + ub_ref[...]
    x = h_ref[0] + u
    m = jnp.mean(x, axis=1, keepdims=True)
    xc = x - m
    v = jnp.mean(xc * xc, axis=1, keepdims=True)
    out_ref[0] = xc * lax.rsqrt(v + 1e-5) * lg_ref[...] + lb_ref[...]


def kernel(edge_index, rels, scores, edge_conf_mask, edge_mask, mask,
           r_query_embed, conf_embeds, rel_embed_table, beta_w, beta_b,
           msg_w, msg_b, upd_w, upd_b, ln_g, ln_b):
    Bn, MaxN = mask.shape
    E = rels.shape[1]
    pe = E_PAD - E
    n_tiles = NC * NS

    src = jnp.pad(edge_index[:, 0, :], ((0, 0), (0, pe)))
    tgt = jnp.pad(edge_index[:, 1, :], ((0, 0), (0, pe)))
    rels_p = jnp.pad(rels, ((0, 0), (0, pe)))
    sc_c = jnp.pad(scores, ((0, 0), (0, pe)))[..., None]
    cm_c = jnp.pad(edge_conf_mask.astype(jnp.int32), ((0, 0), (0, pe)))[..., None]
    em_c = jnp.pad(edge_mask.astype(jnp.float32), ((0, 0), (0, pe)))[..., None]
    src_c = src[..., None]
    tgt_c = tgt[..., None]
    cf = jnp.pad(conf_embeds, ((0, 0), (0, pe), (0, 0))).astype(jnp.bfloat16)
    rq = r_query_embed[:, None, :]
    bw_row = beta_w.reshape(1, D)
    bb = beta_b.reshape(1, 1)

    rep = jnp.broadcast_to(
        rel_embed_table[None], (n_tiles,) + rel_embed_table.shape)
    rep = rep.reshape(-1, D)
    MA = Bn * E_PAD
    rpt = MA // n_tiles
    tile_off = (jnp.arange(MA, dtype=jnp.int32) // rpt) * N_RELS
    boff = jnp.arange(PB, dtype=jnp.int32)[:, None] * N_PAD

    w1 = msg_w[:, 0:D, :]
    w2s = jnp.sum(msg_w[:, D:2 * D, :], axis=1, keepdims=True)
    w3s = jnp.sum(msg_w[:, 2 * D:3 * D, :], axis=1, keepdims=True)
    w14_0 = jnp.concatenate([w1[0], msg_w[0, 3 * D:4 * D, :]], axis=1)
    w5_0 = msg_w[0, 4 * D:5 * D, :].astype(jnp.bfloat16)
    w12 = jnp.concatenate([w1, msg_w[:, D:2 * D, :]], axis=1)
    w45bf = msg_w[:, 3 * D:5 * D, :].astype(jnp.bfloat16)
    mb = msg_b[:, None, :]
    ub = upd_b[:, None, :]
    lg = ln_g.reshape(1, D)
    lb = ln_b.reshape(1, D)

    feat = pl.BlockSpec((1, R, D), lambda b, g: (b, g, 0))
    col = pl.BlockSpec((1, R, 1), lambda b, g: (b, g, 0))
    full2 = lambda shape: pl.BlockSpec(shape, lambda b, g: (0, 0))
    grid = (PB, E_PAD // R)
    node = pl.BlockSpec((1, RN, D), lambda b, g: (b, g, 0))
    node_grid = (PB, N_PAD // RN)
    rq_spec = pl.BlockSpec((1, 1, D), lambda b, g: (b, 0, 0))
    acc_spec = pl.BlockSpec((1, 1, D), lambda b, g: (b, 0, 0))

    tok = jnp.zeros((8,), jnp.int32)
    npair = Bn // PB
    rels_f = [None] * npair
    src_g = [None] * npair
    tgt_f = [None] * npair
    sls = [slice(q * PB, q * PB + PB) for q in range(npair)]
    for q in range(npair):
        src_g[q] = (src[sls[q]] + boff).reshape(-1)
        tgt_f[q] = tgt[sls[q]].reshape(-1)

    rels_all = rels_p.reshape(-1) + tile_off
    hr = [None] * npair
    for q in range(npair):
        half = Bn * E_PAD // npair
        hrq, tok = _sc_gather_rows(
            rep, rels_all[q * half:(q + 1) * half], tok)
        hr[q] = hrq.reshape(PB, E_PAD, D)

    msg0, gate, hr_bf = [None] * npair, [None] * npair, [None] * npair
    for q in range(npair):
        msg0[q], gate[q], hr_bf[q] = pl.pallas_call(
            _msg0_body,
            grid=grid,
            in_specs=[feat, feat, col, col, col, col, rq_spec,
                      full2((1, D)), full2((1, 1)), full2((D, 2 * D)),
                      full2((D, D)), full2((1, D)), full2((1, D))],
            out_specs=[feat, col, feat],
            out_shape=[jax.ShapeDtypeStruct((PB, E_PAD, D), jnp.float32),
                       jax.ShapeDtypeStruct((PB, E_PAD, 1), jnp.float32),
                       jax.ShapeDtypeStruct((PB, E_PAD, D), jnp.bfloat16)],
        )(hr[q], cf[sls[q]], sc_c[sls[q]], cm_c[sls[q]], em_c[sls[q]],
          src_c[sls[q]], rq[sls[q]], bw_row, bb, w14_0, w5_0,
          w2s[0] + w3s[0], mb[0])

    aggr0 = [None] * npair
    for q in range(npair):
        aggr0[q], tok = _sc_scatter_add(msg0[q].reshape(-1, D), tgt_f[q], tok)

    h1 = [None] * npair
    for q in range(npair):
        h1[q] = pl.pallas_call(
            _upd0_body,
            grid=node_grid,
            in_specs=[node, full2((D, D)), full2((1, D)),
                      full2((1, D)), full2((1, D))],
            out_specs=node,
            out_shape=jax.ShapeDtypeStruct((PB, N_PAD, D), jnp.float32),
        )(aggr0[q], upd_w[0], ub[0], lg, lb)

    hs1 = [None] * npair
    for q in range(npair):
        hs1[q], tok = _sc_gather_rows(h1[q].reshape(-1, D), src_g[q], tok)
        hs1[q] = hs1[q].reshape(PB, E_PAD, D)

    msg1 = [None] * npair
    for q in range(npair):
        msg1[q] = pl.pallas_call(
            _msgk_body,
            grid=grid,
            in_specs=[feat, feat, feat, col, col,
                      full2((2 * D, D)), full2((2 * D, D)), full2((1, D)),
                      full2((1, D))],
            out_specs=feat,
            out_shape=jax.ShapeDtypeStruct((PB, E_PAD, D), jnp.float32),
        )(hs1[q], hr_bf[q], cf[sls[q]], gate[q], src_c[sls[q]],
          w12[1], w45bf[1], w3s[1], mb[1])

    aggr1 = [None] * npair
    for q in range(npair):
        aggr1[q], tok = _sc_scatter_add(msg1[q].reshape(-1, D), tgt_f[q], tok)

    h2 = [None] * npair
    for q in range(npair):
        h2[q] = pl.pallas_call(
            _upd_body,
            grid=node_grid,
            in_specs=[node, node, full2((D, D)), full2((1, D)),
                      full2((1, D)), full2((1, D))],
            out_specs=node,
            out_shape=jax.ShapeDtypeStruct((PB, N_PAD, D), jnp.float32),
        )(aggr1[q], h1[q], upd_w[1], ub[1], lg, lb)

    hs2 = [None] * npair
    for q in range(npair):
        hs2[q], tok = _sc_gather_rows(h2[q].reshape(-1, D), src_g[q], tok)
        hs2[q] = hs2[q].reshape(PB, E_PAD, D)

    ctx = []
    for q in range(npair):
        ctx2 = pl.pallas_call(
            _msg2_body,
            grid=grid,
            in_specs=[feat, feat, feat, col, col, col, acc_spec,
                      full2((2 * D, D)), full2((2 * D, D)), full2((1, D)),
                      full2((1, D)), full2((D, D)), full2((1, D)),
                      full2((1, D)), full2((1, D))],
            out_specs=acc_spec,
            out_shape=jax.ShapeDtypeStruct((PB, 1, D), jnp.float32),
        )(hs2[q], hr_bf[q], cf[sls[q]], gate[q], src_c[sls[q]],
          tgt_c[sls[q]], h2[q][:, :1, :],
          w12[2], w45bf[2], w3s[2], mb[2], upd_w[2], ub[2], lg, lb)
        ctx.append(jnp.stack(
            [h1[q][:, 0, :], h2[q][:, 0, :], ctx2[:, 0, :]], axis=1))

    return jnp.concatenate(ctx, axis=0)
